# baseline (speedup 1.0000x reference)
import functools

import jax
import jax.numpy as jnp
from jax import lax
from jax.experimental import pallas as pl
from jax.experimental.pallas import tpu as pltpu
from jax.experimental.pallas import tpu_sc as plsc

B, S, H, E, K, FF = 1, 2048, 1024, 8, 2, 4096
T = B * S
C = (K * T + E - 1) // E
TB = 128
NB = T // TB
FB = 512
NF = FF // FB


def _gate_body(x_ref, g_ref, b_ref, wg_ref, normed_ref, info_ref, stats_ref,
               carry):
    i = pl.program_id(0)
    x = x_ref[...]
    mu = jnp.mean(x, axis=-1, keepdims=True)
    xc = x - mu
    var = jnp.mean(xc * xc, axis=-1, keepdims=True)
    normed = xc * lax.rsqrt(var + 1e-5) * g_ref[...] + b_ref[...]
    normed_ref[...] = normed

    logits = jnp.dot(normed, wg_ref[...], preferred_element_type=jnp.float32)
    m = jnp.max(logits, axis=-1, keepdims=True)
    ex = jnp.exp(logits - m)
    gates = ex / jnp.sum(ex, axis=-1, keepdims=True)

    iota = lax.broadcasted_iota(jnp.int32, (TB, E), 1).astype(jnp.float32)
    v0 = jnp.max(gates, axis=-1, keepdims=True)
    e0 = jnp.min(jnp.where(gates == v0, iota, float(E)), axis=-1,
                 keepdims=True)
    mask0 = (iota == e0).astype(jnp.float32)
    g2 = jnp.where(mask0 > 0, -1.0, gates)
    v1 = jnp.max(g2, axis=-1, keepdims=True)
    e1 = jnp.min(jnp.where(g2 == v1, iota, float(E)), axis=-1, keepdims=True)
    mask1 = (iota == e1).astype(jnp.float32)
    denom = jnp.maximum(v0 + v1, 1e-9)
    gk0 = v0 / denom
    gk1 = v1 / denom

    r_io = lax.broadcasted_iota(jnp.int32, (TB, TB), 0)
    c_io = lax.broadcasted_iota(jnp.int32, (TB, TB), 1)
    tri = (c_io < r_io).astype(jnp.float32)
    excl0 = jnp.dot(tri, mask0, preferred_element_type=jnp.float32)
    excl1 = jnp.dot(tri, mask1, preferred_element_type=jnp.float32)

    @pl.when(i == 0)
    def _():
        carry[...] = jnp.zeros_like(carry)

    carry0 = carry[0:1, :]
    carry1 = carry[1:2, :]
    loc0 = jnp.sum((excl0 + carry0) * mask0, axis=-1, keepdims=True)
    loc1 = jnp.sum((excl1 + carry1) * mask1, axis=-1, keepdims=True)
    carry[0:1, :] = carry0 + jnp.sum(mask0, axis=0, keepdims=True)
    carry[1:2, :] = carry1 + jnp.sum(mask1, axis=0, keepdims=True)
    carry[2:3, :] = (jnp.where(i == 0, 0.0, carry[2:3, :])
                     + jnp.sum(gates, axis=0, keepdims=True))

    sel = lambda j: (iota == float(j)).astype(jnp.float32)
    info_ref[...] = (e0 * sel(0) + e1 * sel(1) + loc0 * sel(2)
                     + loc1 * sel(3) + gk0 * sel(4) + gk1 * sel(5))

    @pl.when(i == NB - 1)
    def _():
        count0 = carry[0:1, :]
        count1 = carry[1:2, :]
        sumg = carry[2:3, :]
        total = count0 + count1
        l_aux = (float(E) / (T * T)) * jnp.sum(sumg * count0)
        io8 = lax.broadcasted_iota(jnp.int32, (1, E), 1).astype(jnp.float32)
        mn = jnp.min(total)
        estar = jnp.min(jnp.where(total == mn, io8, float(E)))
        dummyf = estar * C + (C - 1)
        r_io8 = lax.broadcasted_iota(jnp.int32, (E, E), 0)
        c_io8 = lax.broadcasted_iota(jnp.int32, (E, E), 1)
        stats = (jnp.where(r_io8 == 0, jnp.broadcast_to(count0, (E, E)), 0.0)
                 + jnp.where(r_io8 == 1, jnp.broadcast_to(total, (E, E)), 0.0)
                 + jnp.where((r_io8 == 3) & (c_io8 == 0), l_aux, 0.0)
                 + jnp.where((r_io8 == 3) & (c_io8 == 1), dummyf, 0.0))
        stats_ref[...] = stats


def _gate_call(xf, gamma, beta, Wg):
    return pl.pallas_call(
        _gate_body,
        grid=(NB,),
        in_specs=[
            pl.BlockSpec((TB, H), lambda i: (i, 0)),
            pl.BlockSpec((1, H), lambda i: (0, 0)),
            pl.BlockSpec((1, H), lambda i: (0, 0)),
            pl.BlockSpec((H, E), lambda i: (0, 0)),
        ],
        out_specs=[
            pl.BlockSpec((TB, H), lambda i: (i, 0)),
            pl.BlockSpec((TB, E), lambda i: (i, 0)),
            pl.BlockSpec((E, E), lambda i: (0, 0)),
        ],
        out_shape=[
            jax.ShapeDtypeStruct((T, H), jnp.float32),
            jax.ShapeDtypeStruct((T, E), jnp.float32),
            jax.ShapeDtypeStruct((E, E), jnp.float32),
        ],
        scratch_shapes=[pltpu.VMEM((E, E), jnp.float32)],
        compiler_params=pltpu.CompilerParams(
            dimension_semantics=("arbitrary",)),
    )(xf, gamma, beta, Wg)


def _ffn_body(x_ref, w1_ref, b1_ref, w2_ref, b2_ref, gks_ref, y_ref):
    f = pl.program_id(1)
    x = x_ref[0].astype(jnp.bfloat16)
    h = jnp.dot(x, w1_ref[0], preferred_element_type=jnp.float32) + b1_ref[0]
    h = 0.5 * h * (1.0 + lax.erf(h * 0.7071067811865476))
    contrib = jnp.dot(h.astype(jnp.bfloat16), w2_ref[0],
                      preferred_element_type=jnp.float32)

    @pl.when(f == 0)
    def _():
        y_ref[0] = contrib

    @pl.when(f > 0)
    def _():
        y_ref[0] = y_ref[0] + contrib

    @pl.when(f == NF - 1)
    def _():
        y_ref[0] = (y_ref[0] + b2_ref[0]) * gks_ref[0]


def _ffn_call(expert_in, W1, b1, W2, b2, gks):
    W1 = W1.astype(jnp.bfloat16)
    W2 = W2.astype(jnp.bfloat16)
    return pl.pallas_call(
        _ffn_body,
        grid=(E, NF),
        in_specs=[
            pl.BlockSpec((1, C, H), lambda e, f: (e, 0, 0)),
            pl.BlockSpec((1, H, FB), lambda e, f: (e, 0, f)),
            pl.BlockSpec((1, 1, FB), lambda e, f: (e * NF + f, 0, 0)),
            pl.BlockSpec((1, FB, H), lambda e, f: (e, f, 0)),
            pl.BlockSpec((1, 1, H), lambda e, f: (e, 0, 0)),
            pl.BlockSpec((1, C, 1), lambda e, f: (e, 0, 0)),
        ],
        out_specs=pl.BlockSpec((1, C, H), lambda e, f: (e, 0, 0)),
        out_shape=jax.ShapeDtypeStruct((E, C, H), jnp.float32),
        compiler_params=pltpu.CompilerParams(
            dimension_semantics=("parallel", "arbitrary")),
    )(expert_in, W1, b1.reshape(E * NF, 1, FB), W2, b2.reshape(E, 1, H), gks)


NC, NS, L = 2, 16, 16
NW = NC * NS
TPW = T // NW
SPW = (E * C) // NW
_MESH = plsc.VectorSubcoreMesh(core_axis_name="c", subcore_axis_name="s",
                               num_cores=NC, num_subcores=NS)


def _dispatch_sc(e0, e1, loc0, loc1r, gk0, gk1, cnt0, dmy, normed):
    @functools.partial(
        pl.kernel,
        out_type=[
            jax.ShapeDtypeStruct((E * C, H), jnp.float32),
            jax.ShapeDtypeStruct((E * C,), jnp.float32),
            jax.ShapeDtypeStruct((T,), jnp.int32),
            jax.ShapeDtypeStruct((T,), jnp.int32),
        ],
        mesh=_MESH,
        scratch_types=[
            pltpu.VMEM((T,), jnp.int32),
            pltpu.VMEM((T,), jnp.int32),
            pltpu.VMEM((T,), jnp.int32),
            pltpu.VMEM((T,), jnp.int32),
            pltpu.VMEM((T,), jnp.float32),
            pltpu.VMEM((T,), jnp.float32),
            pltpu.VMEM((16,), jnp.int32),
            pltpu.VMEM((16,), jnp.int32),
            pltpu.VMEM((E * C,), jnp.int32),
            pltpu.VMEM((E * C,), jnp.float32),
            pltpu.VMEM((T,), jnp.int32),
            pltpu.VMEM((T,), jnp.int32),
            pltpu.VMEM_SHARED((E * C,), jnp.int32),
            pltpu.VMEM((64,), jnp.int32),
            pltpu.VMEM((64, H), jnp.float32),
            pltpu.SemaphoreType.DMA,
        ],
        compiler_params=pltpu.CompilerParams(needs_layout_passes=False),
    )
    def body(e0_h, e1_h, l0_h, l1_h, g0_h, g1_h, c0_h, dm_h, nm_h,
             ei_h, gs_h, d0_h, d1_h,
             e0v, e1v, l0v, l1v, g0v, g1v, cntv, dmyv, stv, gsv, d0v, d1v,
             shst, idxv, rows, sem):
        cid = lax.axis_index("c")
        sid = lax.axis_index("s")
        wid = sid * NC + cid

        @pl.when(sid == 0)
        def _phase1():
            pltpu.sync_copy(e0_h, e0v)
            pltpu.sync_copy(e1_h, e1v)
            pltpu.sync_copy(l0_h, l0v)
            pltpu.sync_copy(l1_h, l1v)
            pltpu.sync_copy(g0_h, g0v)
            pltpu.sync_copy(g1_h, g1v)
            pltpu.sync_copy(c0_h, cntv)
            pltpu.sync_copy(dm_h, dmyv)

            def zinit(j, _):
                stv[pl.ds(j * L, L)] = jnp.zeros((L,), jnp.int32)
                gsv[pl.ds(j * L, L)] = jnp.zeros((L,), jnp.float32)
                return 0
            lax.fori_loop(0, (E * C) // L, zinit, 0)

            dmy16 = dmyv[...]

            def route(g, _):
                base = g * L
                tvec = lax.iota(jnp.int32, L) + base
                e0g = e0v[pl.ds(base, L)]
                l0g = l0v[pl.ds(base, L)]
                d0 = e0g * C + l0g
                m0 = l0g < C
                plsc.store_scatter(stv, [d0], tvec, mask=m0)
                plsc.store_scatter(gsv, [d0], g0v[pl.ds(base, L)], mask=m0)
                e1g = e1v[pl.ds(base, L)]
                c0g = plsc.load_gather(cntv, [e1g])
                s1 = l1v[pl.ds(base, L)] + c0g
                d1 = e1g * C + s1
                m1 = s1 < C
                plsc.store_scatter(stv, [d1], tvec, mask=m1)
                plsc.store_scatter(gsv, [d1], g1v[pl.ds(base, L)], mask=m1)
                d0v[pl.ds(base, L)] = jnp.where(m0, d0, dmy16)
                d1v[pl.ds(base, L)] = jnp.where(m1, d1, dmy16)
                return 0
            lax.fori_loop(0, T // L, route, 0)

            pltpu.sync_copy(stv, shst)

            @pl.when(cid == 0)
            def _():
                pltpu.sync_copy(gsv, gs_h)
                pltpu.sync_copy(d0v, d0_h)
                pltpu.sync_copy(d1v, d1_h)

        plsc.subcore_barrier()

        base = wid * SPW
        for j in range(SPW // 64):
            pltpu.sync_copy(shst.at[pl.ds(base + j * 64, 64)], idxv)
            pltpu.async_copy(nm_h.at[idxv], rows, sem).wait()
            pltpu.sync_copy(rows, ei_h.at[pl.ds(base + j * 64, 64)])

    return body(e0, e1, loc0, loc1r, gk0, gk1, cnt0, dmy, normed)


def _combine_sc(xf, ys, d0m, d1m):
    CH = 16

    @functools.partial(
        pl.kernel,
        out_type=jax.ShapeDtypeStruct((T, H), jnp.float32),
        mesh=_MESH,
        scratch_types=[
            pltpu.VMEM((CH,), jnp.int32),
            pltpu.VMEM((CH,), jnp.int32),
            pltpu.VMEM((CH, H), jnp.float32),
            pltpu.VMEM((CH, H), jnp.float32),
            pltpu.VMEM((CH, H), jnp.float32),
            pltpu.SemaphoreType.DMA,
        ],
    )
    def body(x_h, ys_h, d0_h, d1_h, o_h, i0v, i1v, xv, r0v, r1v, sem):
        cid = lax.axis_index("c")
        sid = lax.axis_index("s")
        wid = sid * NC + cid
        for ck in range(TPW // CH):
            tb = wid * TPW + ck * CH
            pltpu.sync_copy(d0_h.at[pl.ds(tb, CH)], i0v)
            pltpu.sync_copy(d1_h.at[pl.ds(tb, CH)], i1v)
            pltpu.sync_copy(x_h.at[pl.ds(tb, CH)], xv)
            cp0 = pltpu.async_copy(ys_h.at[i0v], r0v, sem)
            cp1 = pltpu.async_copy(ys_h.at[i1v], r1v, sem)
            cp0.wait()
            cp1.wait()

            def row(r, _):
                def col(j, _):
                    xv[r, pl.ds(j * L, L)] = (xv[r, pl.ds(j * L, L)]
                                              + r0v[r, pl.ds(j * L, L)]
                                              + r1v[r, pl.ds(j * L, L)])
                    return 0
                lax.fori_loop(0, H // L, col, 0)
                return 0
            lax.fori_loop(0, CH, row, 0)
            pltpu.sync_copy(xv, o_h.at[pl.ds(tb, CH)])

    return body(xf, ys, d0m, d1m)


def kernel(x, gamma, beta, Wg, W1, b1, W2, b2):
    xf = x.reshape(T, H)
    normed, info, stats = _gate_call(xf, gamma.reshape(1, H),
                                     beta.reshape(1, H), Wg)
    e0 = info[:, 0].astype(jnp.int32)
    e1 = info[:, 1].astype(jnp.int32)
    loc0 = info[:, 2].astype(jnp.int32)
    loc1r = info[:, 3].astype(jnp.int32)
    gk0 = info[:, 4]
    gk1 = info[:, 5]
    counts = stats[1]
    l_aux = stats[3, 0]
    cnt0 = jnp.concatenate([stats[0], jnp.zeros((8,), jnp.float32)]
                           ).astype(jnp.int32)
    dmy = jnp.broadcast_to(stats[3, 1], (16,)).astype(jnp.int32)

    expert_in, gk_slot, d0m, d1m = _dispatch_sc(e0, e1, loc0, loc1r,
                                                gk0, gk1, cnt0, dmy, normed)
    ys = _ffn_call(expert_in.reshape(E, C, H), W1, b1, W2, b2,
                   gk_slot.reshape(E, C, 1)).reshape(E * C, H)
    out_flat = _combine_sc(xf, ys, d0m, d1m)
    return out_flat.reshape(B, S, H), l_aux, counts


# speedup vs baseline: 1.3536x; 1.3536x over previous
import functools

import jax
import jax.numpy as jnp
from jax import lax
from jax.experimental import pallas as pl
from jax.experimental.pallas import tpu as pltpu
from jax.experimental.pallas import tpu_sc as plsc

B, S, H, E, K, FF = 1, 2048, 1024, 8, 2, 4096
T = B * S
C = (K * T + E - 1) // E
TB = 128
NB = T // TB
FB = 512
NF = FF // FB


def _gate_body(x_ref, g_ref, b_ref, wg_ref, normed_ref, info_ref, stats_ref,
               carry):
    i = pl.program_id(0)
    x = x_ref[...]
    mu = jnp.mean(x, axis=-1, keepdims=True)
    xc = x - mu
    var = jnp.mean(xc * xc, axis=-1, keepdims=True)
    normed = xc * lax.rsqrt(var + 1e-5) * g_ref[...] + b_ref[...]
    normed_ref[...] = normed

    logits = jnp.dot(normed, wg_ref[...], preferred_element_type=jnp.float32)
    m = jnp.max(logits, axis=-1, keepdims=True)
    ex = jnp.exp(logits - m)
    gates = ex / jnp.sum(ex, axis=-1, keepdims=True)

    iota = lax.broadcasted_iota(jnp.int32, (TB, E), 1).astype(jnp.float32)
    v0 = jnp.max(gates, axis=-1, keepdims=True)
    e0 = jnp.min(jnp.where(gates == v0, iota, float(E)), axis=-1,
                 keepdims=True)
    mask0 = (iota == e0).astype(jnp.float32)
    g2 = jnp.where(mask0 > 0, -1.0, gates)
    v1 = jnp.max(g2, axis=-1, keepdims=True)
    e1 = jnp.min(jnp.where(g2 == v1, iota, float(E)), axis=-1, keepdims=True)
    mask1 = (iota == e1).astype(jnp.float32)
    denom = jnp.maximum(v0 + v1, 1e-9)
    gk0 = v0 / denom
    gk1 = v1 / denom

    r_io = lax.broadcasted_iota(jnp.int32, (TB, TB), 0)
    c_io = lax.broadcasted_iota(jnp.int32, (TB, TB), 1)
    tri = (c_io < r_io).astype(jnp.float32)
    excl0 = jnp.dot(tri, mask0, preferred_element_type=jnp.float32)
    excl1 = jnp.dot(tri, mask1, preferred_element_type=jnp.float32)

    @pl.when(i == 0)
    def _():
        carry[...] = jnp.zeros_like(carry)

    carry0 = carry[0:1, :]
    carry1 = carry[1:2, :]
    loc0 = jnp.sum((excl0 + carry0) * mask0, axis=-1, keepdims=True)
    loc1 = jnp.sum((excl1 + carry1) * mask1, axis=-1, keepdims=True)
    carry[0:1, :] = carry0 + jnp.sum(mask0, axis=0, keepdims=True)
    carry[1:2, :] = carry1 + jnp.sum(mask1, axis=0, keepdims=True)
    carry[2:3, :] = (jnp.where(i == 0, 0.0, carry[2:3, :])
                     + jnp.sum(gates, axis=0, keepdims=True))

    sel = lambda j: (iota == float(j)).astype(jnp.float32)
    info_ref[...] = (e0 * sel(0) + e1 * sel(1) + loc0 * sel(2)
                     + loc1 * sel(3) + gk0 * sel(4) + gk1 * sel(5))

    @pl.when(i == NB - 1)
    def _():
        count0 = carry[0:1, :]
        count1 = carry[1:2, :]
        sumg = carry[2:3, :]
        total = count0 + count1
        l_aux = (float(E) / (T * T)) * jnp.sum(sumg * count0)
        io8 = lax.broadcasted_iota(jnp.int32, (1, E), 1).astype(jnp.float32)
        mn = jnp.min(total)
        estar = jnp.min(jnp.where(total == mn, io8, float(E)))
        dummyf = estar * C + (C - 1)
        r_io8 = lax.broadcasted_iota(jnp.int32, (E, E), 0)
        c_io8 = lax.broadcasted_iota(jnp.int32, (E, E), 1)
        stats = (jnp.where(r_io8 == 0, jnp.broadcast_to(count0, (E, E)), 0.0)
                 + jnp.where(r_io8 == 1, jnp.broadcast_to(total, (E, E)), 0.0)
                 + jnp.where((r_io8 == 3) & (c_io8 == 0), l_aux, 0.0)
                 + jnp.where((r_io8 == 3) & (c_io8 == 1), dummyf, 0.0))
        stats_ref[...] = stats


def _gate_call(xf, gamma, beta, Wg):
    return pl.pallas_call(
        _gate_body,
        grid=(NB,),
        in_specs=[
            pl.BlockSpec((TB, H), lambda i: (i, 0)),
            pl.BlockSpec((1, H), lambda i: (0, 0)),
            pl.BlockSpec((1, H), lambda i: (0, 0)),
            pl.BlockSpec((H, E), lambda i: (0, 0)),
        ],
        out_specs=[
            pl.BlockSpec((TB, H), lambda i: (i, 0)),
            pl.BlockSpec((TB, E), lambda i: (i, 0)),
            pl.BlockSpec((E, E), lambda i: (0, 0)),
        ],
        out_shape=[
            jax.ShapeDtypeStruct((T, H), jnp.float32),
            jax.ShapeDtypeStruct((T, E), jnp.float32),
            jax.ShapeDtypeStruct((E, E), jnp.float32),
        ],
        scratch_shapes=[pltpu.VMEM((E, E), jnp.float32)],
        compiler_params=pltpu.CompilerParams(
            dimension_semantics=("arbitrary",)),
    )(xf, gamma, beta, Wg)


def _ffn_body(x_ref, w1_ref, b1_ref, w2_ref, b2_ref, gks_ref, y_ref):
    f = pl.program_id(1)
    x = x_ref[0].astype(jnp.bfloat16)
    h = jnp.dot(x, w1_ref[0].astype(jnp.bfloat16),
                preferred_element_type=jnp.float32) + b1_ref[0]
    h = 0.5 * h * (1.0 + lax.erf(h * 0.7071067811865476))
    contrib = jnp.dot(h.astype(jnp.bfloat16), w2_ref[0].astype(jnp.bfloat16),
                      preferred_element_type=jnp.float32)

    @pl.when(f == 0)
    def _():
        y_ref[0] = contrib

    @pl.when(f > 0)
    def _():
        y_ref[0] = y_ref[0] + contrib

    @pl.when(f == NF - 1)
    def _():
        y_ref[0] = (y_ref[0] + b2_ref[0]) * gks_ref[0]


def _ffn_call(expert_in, W1, b1, W2, b2, gks):
    return pl.pallas_call(
        _ffn_body,
        grid=(E, NF),
        in_specs=[
            pl.BlockSpec((1, C, H), lambda e, f: (e, 0, 0)),
            pl.BlockSpec((1, H, FB), lambda e, f: (e, 0, f)),
            pl.BlockSpec((1, 1, FB), lambda e, f: (e * NF + f, 0, 0)),
            pl.BlockSpec((1, FB, H), lambda e, f: (e, f, 0)),
            pl.BlockSpec((1, 1, H), lambda e, f: (e, 0, 0)),
            pl.BlockSpec((1, C, 1), lambda e, f: (e, 0, 0)),
        ],
        out_specs=pl.BlockSpec((1, C, H), lambda e, f: (e, 0, 0)),
        out_shape=jax.ShapeDtypeStruct((E, C, H), jnp.float32),
        compiler_params=pltpu.CompilerParams(
            dimension_semantics=("parallel", "arbitrary")),
    )(expert_in, W1, b1.reshape(E * NF, 1, FB), W2, b2.reshape(E, 1, H), gks)


NC, NS, L = 2, 16, 16
NW = NC * NS
TPW = T // NW
SPW = (E * C) // NW
_MESH = plsc.VectorSubcoreMesh(core_axis_name="c", subcore_axis_name="s",
                               num_cores=NC, num_subcores=NS)


def _dispatch_sc(e0, e1, loc0, loc1r, gk0, gk1, cnt0, dmy, normed):
    @functools.partial(
        pl.kernel,
        out_type=[
            jax.ShapeDtypeStruct((E * C, H), jnp.float32),
            jax.ShapeDtypeStruct((E * C,), jnp.float32),
            jax.ShapeDtypeStruct((T,), jnp.int32),
            jax.ShapeDtypeStruct((T,), jnp.int32),
        ],
        mesh=_MESH,
        scratch_types=[
            pltpu.VMEM((T,), jnp.int32),
            pltpu.VMEM((T,), jnp.int32),
            pltpu.VMEM((T,), jnp.int32),
            pltpu.VMEM((T,), jnp.int32),
            pltpu.VMEM((T,), jnp.float32),
            pltpu.VMEM((T,), jnp.float32),
            pltpu.VMEM((16,), jnp.int32),
            pltpu.VMEM((16,), jnp.int32),
            pltpu.VMEM((E * C,), jnp.int32),
            pltpu.VMEM((E * C,), jnp.float32),
            pltpu.VMEM((T,), jnp.int32),
            pltpu.VMEM((T,), jnp.int32),
            pltpu.VMEM_SHARED((E * C,), jnp.int32),
            pltpu.VMEM((64,), jnp.int32),
            pltpu.VMEM((64, H), jnp.float32),
            pltpu.SemaphoreType.DMA,
        ],
        compiler_params=pltpu.CompilerParams(needs_layout_passes=False),
    )
    def body(e0_h, e1_h, l0_h, l1_h, g0_h, g1_h, c0_h, dm_h, nm_h,
             ei_h, gs_h, d0_h, d1_h,
             e0v, e1v, l0v, l1v, g0v, g1v, cntv, dmyv, stv, gsv, d0v, d1v,
             shst, idxv, rows, sem):
        cid = lax.axis_index("c")
        sid = lax.axis_index("s")
        wid = sid * NC + cid

        @pl.when(sid == 0)
        def _phase1():
            pltpu.sync_copy(e0_h, e0v)
            pltpu.sync_copy(e1_h, e1v)
            pltpu.sync_copy(l0_h, l0v)
            pltpu.sync_copy(l1_h, l1v)
            pltpu.sync_copy(g0_h, g0v)
            pltpu.sync_copy(g1_h, g1v)
            pltpu.sync_copy(c0_h, cntv)
            pltpu.sync_copy(dm_h, dmyv)

            def zinit(j, _):
                stv[pl.ds(j * L, L)] = jnp.zeros((L,), jnp.int32)
                gsv[pl.ds(j * L, L)] = jnp.zeros((L,), jnp.float32)
                return 0
            lax.fori_loop(0, (E * C) // L, zinit, 0)

            dmy16 = dmyv[...]

            def route(g, _):
                base = g * L
                tvec = lax.iota(jnp.int32, L) + base
                e0g = e0v[pl.ds(base, L)]
                l0g = l0v[pl.ds(base, L)]
                d0 = e0g * C + l0g
                m0 = l0g < C
                plsc.store_scatter(stv, [d0], tvec, mask=m0)
                plsc.store_scatter(gsv, [d0], g0v[pl.ds(base, L)], mask=m0)
                e1g = e1v[pl.ds(base, L)]
                c0g = plsc.load_gather(cntv, [e1g])
                s1 = l1v[pl.ds(base, L)] + c0g
                d1 = e1g * C + s1
                m1 = s1 < C
                plsc.store_scatter(stv, [d1], tvec, mask=m1)
                plsc.store_scatter(gsv, [d1], g1v[pl.ds(base, L)], mask=m1)
                d0v[pl.ds(base, L)] = jnp.where(m0, d0, dmy16)
                d1v[pl.ds(base, L)] = jnp.where(m1, d1, dmy16)
                return 0
            lax.fori_loop(0, T // L, route, 0)

            pltpu.sync_copy(stv, shst)

            @pl.when(cid == 0)
            def _():
                pltpu.sync_copy(gsv, gs_h)
                pltpu.sync_copy(d0v, d0_h)
                pltpu.sync_copy(d1v, d1_h)

        plsc.subcore_barrier()

        base = wid * SPW
        for j in range(SPW // 64):
            pltpu.sync_copy(shst.at[pl.ds(base + j * 64, 64)], idxv)
            pltpu.async_copy(nm_h.at[idxv], rows, sem).wait()
            pltpu.sync_copy(rows, ei_h.at[pl.ds(base + j * 64, 64)])

    return body(e0, e1, loc0, loc1r, gk0, gk1, cnt0, dmy, normed)


def _combine_sc(xf, ys, d0m, d1m):
    CH = 16

    @functools.partial(
        pl.kernel,
        out_type=jax.ShapeDtypeStruct((T, H), jnp.float32),
        mesh=_MESH,
        scratch_types=[
            pltpu.VMEM((CH,), jnp.int32),
            pltpu.VMEM((CH,), jnp.int32),
            pltpu.VMEM((CH, H), jnp.float32),
            pltpu.VMEM((CH, H), jnp.float32),
            pltpu.VMEM((CH, H), jnp.float32),
            pltpu.SemaphoreType.DMA,
        ],
    )
    def body(x_h, ys_h, d0_h, d1_h, o_h, i0v, i1v, xv, r0v, r1v, sem):
        cid = lax.axis_index("c")
        sid = lax.axis_index("s")
        wid = sid * NC + cid
        for ck in range(TPW // CH):
            tb = wid * TPW + ck * CH
            pltpu.sync_copy(d0_h.at[pl.ds(tb, CH)], i0v)
            pltpu.sync_copy(d1_h.at[pl.ds(tb, CH)], i1v)
            pltpu.sync_copy(x_h.at[pl.ds(tb, CH)], xv)
            cp0 = pltpu.async_copy(ys_h.at[i0v], r0v, sem)
            cp1 = pltpu.async_copy(ys_h.at[i1v], r1v, sem)
            cp0.wait()
            cp1.wait()

            def row(r, _):
                def col(j, _):
                    xv[r, pl.ds(j * L, L)] = (xv[r, pl.ds(j * L, L)]
                                              + r0v[r, pl.ds(j * L, L)]
                                              + r1v[r, pl.ds(j * L, L)])
                    return 0
                lax.fori_loop(0, H // L, col, 0)
                return 0
            lax.fori_loop(0, CH, row, 0)
            pltpu.sync_copy(xv, o_h.at[pl.ds(tb, CH)])

    return body(xf, ys, d0m, d1m)


def kernel(x, gamma, beta, Wg, W1, b1, W2, b2):
    xf = x.reshape(T, H)
    normed, info, stats = _gate_call(xf, gamma.reshape(1, H),
                                     beta.reshape(1, H), Wg)
    e0 = info[:, 0].astype(jnp.int32)
    e1 = info[:, 1].astype(jnp.int32)
    loc0 = info[:, 2].astype(jnp.int32)
    loc1r = info[:, 3].astype(jnp.int32)
    gk0 = info[:, 4]
    gk1 = info[:, 5]
    counts = stats[1]
    l_aux = stats[3, 0]
    cnt0 = jnp.concatenate([stats[0], jnp.zeros((8,), jnp.float32)]
                           ).astype(jnp.int32)
    dmy = jnp.broadcast_to(stats[3, 1], (16,)).astype(jnp.int32)

    expert_in, gk_slot, d0m, d1m = _dispatch_sc(e0, e1, loc0, loc1r,
                                                gk0, gk1, cnt0, dmy, normed)
    ys = _ffn_call(expert_in.reshape(E, C, H), W1, b1, W2, b2,
                   gk_slot.reshape(E, C, 1)).reshape(E * C, H)
    out_flat = _combine_sc(xf, ys, d0m, d1m)
    return out_flat.reshape(B, S, H), l_aux, counts


# speedup vs baseline: 1.5011x; 1.1089x over previous
import functools

import jax
import jax.numpy as jnp
from jax import lax
from jax.experimental import pallas as pl
from jax.experimental.pallas import tpu as pltpu
from jax.experimental.pallas import tpu_sc as plsc

B, S, H, E, K, FF = 1, 2048, 1024, 8, 2, 4096
T = B * S
C = (K * T + E - 1) // E
TB = 128
NB = T // TB
FB = 1024
NF = FF // FB


def _gate_body(x_ref, g_ref, b_ref, wg_ref, normed_ref, info_ref, stats_ref,
               carry):
    i = pl.program_id(0)
    x = x_ref[...]
    mu = jnp.mean(x, axis=-1, keepdims=True)
    xc = x - mu
    var = jnp.mean(xc * xc, axis=-1, keepdims=True)
    normed = xc * lax.rsqrt(var + 1e-5) * g_ref[...] + b_ref[...]
    normed_ref[...] = normed

    logits = jnp.dot(normed, wg_ref[...], preferred_element_type=jnp.float32)
    m = jnp.max(logits, axis=-1, keepdims=True)
    ex = jnp.exp(logits - m)
    gates = ex / jnp.sum(ex, axis=-1, keepdims=True)

    iota = lax.broadcasted_iota(jnp.int32, (TB, E), 1).astype(jnp.float32)
    v0 = jnp.max(gates, axis=-1, keepdims=True)
    e0 = jnp.min(jnp.where(gates == v0, iota, float(E)), axis=-1,
                 keepdims=True)
    mask0 = (iota == e0).astype(jnp.float32)
    g2 = jnp.where(mask0 > 0, -1.0, gates)
    v1 = jnp.max(g2, axis=-1, keepdims=True)
    e1 = jnp.min(jnp.where(g2 == v1, iota, float(E)), axis=-1, keepdims=True)
    mask1 = (iota == e1).astype(jnp.float32)
    denom = jnp.maximum(v0 + v1, 1e-9)
    gk0 = v0 / denom
    gk1 = v1 / denom

    r_io = lax.broadcasted_iota(jnp.int32, (TB, TB), 0)
    c_io = lax.broadcasted_iota(jnp.int32, (TB, TB), 1)
    tri = (c_io < r_io).astype(jnp.float32)
    excl0 = jnp.dot(tri, mask0, preferred_element_type=jnp.float32)
    excl1 = jnp.dot(tri, mask1, preferred_element_type=jnp.float32)

    @pl.when(i == 0)
    def _():
        carry[...] = jnp.zeros_like(carry)

    carry0 = carry[0:1, :]
    carry1 = carry[1:2, :]
    loc0 = jnp.sum((excl0 + carry0) * mask0, axis=-1, keepdims=True)
    loc1 = jnp.sum((excl1 + carry1) * mask1, axis=-1, keepdims=True)
    carry[0:1, :] = carry0 + jnp.sum(mask0, axis=0, keepdims=True)
    carry[1:2, :] = carry1 + jnp.sum(mask1, axis=0, keepdims=True)
    carry[2:3, :] = (jnp.where(i == 0, 0.0, carry[2:3, :])
                     + jnp.sum(gates, axis=0, keepdims=True))

    sel = lambda j: (iota == float(j)).astype(jnp.float32)
    info_ref[...] = (e0 * sel(0) + e1 * sel(1) + loc0 * sel(2)
                     + loc1 * sel(3) + gk0 * sel(4) + gk1 * sel(5))

    @pl.when(i == NB - 1)
    def _():
        count0 = carry[0:1, :]
        count1 = carry[1:2, :]
        sumg = carry[2:3, :]
        total = count0 + count1
        l_aux = (float(E) / (T * T)) * jnp.sum(sumg * count0)
        io8 = lax.broadcasted_iota(jnp.int32, (1, E), 1).astype(jnp.float32)
        mn = jnp.min(total)
        estar = jnp.min(jnp.where(total == mn, io8, float(E)))
        dummyf = estar * C + (C - 1)
        r_io8 = lax.broadcasted_iota(jnp.int32, (E, E), 0)
        c_io8 = lax.broadcasted_iota(jnp.int32, (E, E), 1)
        stats = (jnp.where(r_io8 == 0, jnp.broadcast_to(count0, (E, E)), 0.0)
                 + jnp.where(r_io8 == 1, jnp.broadcast_to(total, (E, E)), 0.0)
                 + jnp.where((r_io8 == 3) & (c_io8 == 0), l_aux, 0.0)
                 + jnp.where((r_io8 == 3) & (c_io8 == 1), dummyf, 0.0))
        stats_ref[...] = stats


def _gate_call(xf, gamma, beta, Wg):
    return pl.pallas_call(
        _gate_body,
        grid=(NB,),
        in_specs=[
            pl.BlockSpec((TB, H), lambda i: (i, 0)),
            pl.BlockSpec((1, H), lambda i: (0, 0)),
            pl.BlockSpec((1, H), lambda i: (0, 0)),
            pl.BlockSpec((H, E), lambda i: (0, 0)),
        ],
        out_specs=[
            pl.BlockSpec((TB, H), lambda i: (i, 0)),
            pl.BlockSpec((TB, E), lambda i: (i, 0)),
            pl.BlockSpec((E, E), lambda i: (0, 0)),
        ],
        out_shape=[
            jax.ShapeDtypeStruct((T, H), jnp.float32),
            jax.ShapeDtypeStruct((T, E), jnp.float32),
            jax.ShapeDtypeStruct((E, E), jnp.float32),
        ],
        scratch_shapes=[pltpu.VMEM((E, E), jnp.float32)],
        compiler_params=pltpu.CompilerParams(
            dimension_semantics=("arbitrary",)),
    )(xf, gamma, beta, Wg)


def _ffn_body(x_ref, w1_ref, b1_ref, w2_ref, b2_ref, gks_ref, y_ref):
    f = pl.program_id(1)
    x = x_ref[0]
    h = jnp.dot(x, w1_ref[0], preferred_element_type=jnp.float32) + b1_ref[0]
    h = 0.5 * h * (1.0 + lax.erf(h * 0.7071067811865476))
    contrib = jnp.dot(h, w2_ref[0], preferred_element_type=jnp.float32)

    @pl.when(f == 0)
    def _():
        y_ref[0] = contrib

    @pl.when(f > 0)
    def _():
        y_ref[0] = y_ref[0] + contrib

    @pl.when(f == NF - 1)
    def _():
        y_ref[0] = (y_ref[0] + b2_ref[0]) * gks_ref[0]


def _ffn_call(expert_in, W1, b1, W2, b2, gks):
    return pl.pallas_call(
        _ffn_body,
        grid=(E, NF),
        in_specs=[
            pl.BlockSpec((1, C, H), lambda e, f: (e, 0, 0)),
            pl.BlockSpec((1, H, FB), lambda e, f: (e, 0, f)),
            pl.BlockSpec((1, 1, FB), lambda e, f: (e * NF + f, 0, 0)),
            pl.BlockSpec((1, FB, H), lambda e, f: (e, f, 0)),
            pl.BlockSpec((1, 1, H), lambda e, f: (e, 0, 0)),
            pl.BlockSpec((1, C, 1), lambda e, f: (e, 0, 0)),
        ],
        out_specs=pl.BlockSpec((1, C, H), lambda e, f: (e, 0, 0)),
        out_shape=jax.ShapeDtypeStruct((E, C, H), jnp.float32),
        compiler_params=pltpu.CompilerParams(
            dimension_semantics=("parallel", "arbitrary")),
    )(expert_in, W1, b1.reshape(E * NF, 1, FB), W2, b2.reshape(E, 1, H), gks)


NC, NS, L = 2, 16, 16
NW = NC * NS
TPW = T // NW
SPW = (E * C) // NW
_MESH = plsc.VectorSubcoreMesh(core_axis_name="c", subcore_axis_name="s",
                               num_cores=NC, num_subcores=NS)


def _dispatch_sc(e0, e1, loc0, loc1r, gk0, gk1, cnt0, dmy, normed):
    @functools.partial(
        pl.kernel,
        out_type=[
            jax.ShapeDtypeStruct((E * C, H), jnp.float32),
            jax.ShapeDtypeStruct((E * C,), jnp.float32),
            jax.ShapeDtypeStruct((T,), jnp.int32),
            jax.ShapeDtypeStruct((T,), jnp.int32),
        ],
        mesh=_MESH,
        scratch_types=[
            pltpu.VMEM((T,), jnp.int32),
            pltpu.VMEM((T,), jnp.int32),
            pltpu.VMEM((T,), jnp.int32),
            pltpu.VMEM((T,), jnp.int32),
            pltpu.VMEM((T,), jnp.float32),
            pltpu.VMEM((T,), jnp.float32),
            pltpu.VMEM((16,), jnp.int32),
            pltpu.VMEM((16,), jnp.int32),
            pltpu.VMEM((E * C,), jnp.int32),
            pltpu.VMEM((E * C,), jnp.float32),
            pltpu.VMEM((T,), jnp.int32),
            pltpu.VMEM((T,), jnp.int32),
            pltpu.VMEM_SHARED((E * C,), jnp.int32),
            pltpu.VMEM((64,), jnp.int32),
            pltpu.VMEM((64, H), jnp.float32),
            pltpu.SemaphoreType.DMA,
        ],
        compiler_params=pltpu.CompilerParams(needs_layout_passes=False),
    )
    def body(e0_h, e1_h, l0_h, l1_h, g0_h, g1_h, c0_h, dm_h, nm_h,
             ei_h, gs_h, d0_h, d1_h,
             e0v, e1v, l0v, l1v, g0v, g1v, cntv, dmyv, stv, gsv, d0v, d1v,
             shst, idxv, rows, sem):
        cid = lax.axis_index("c")
        sid = lax.axis_index("s")
        wid = sid * NC + cid

        @pl.when(sid == 0)
        def _phase1():
            pltpu.sync_copy(e0_h, e0v)
            pltpu.sync_copy(e1_h, e1v)
            pltpu.sync_copy(l0_h, l0v)
            pltpu.sync_copy(l1_h, l1v)
            pltpu.sync_copy(g0_h, g0v)
            pltpu.sync_copy(g1_h, g1v)
            pltpu.sync_copy(c0_h, cntv)
            pltpu.sync_copy(dm_h, dmyv)

            def zinit(j, _):
                stv[pl.ds(j * L, L)] = jnp.zeros((L,), jnp.int32)
                gsv[pl.ds(j * L, L)] = jnp.zeros((L,), jnp.float32)
                return 0
            lax.fori_loop(0, (E * C) // L, zinit, 0)

            dmy16 = dmyv[...]

            def route(g, _):
                base = g * L
                tvec = lax.iota(jnp.int32, L) + base
                e0g = e0v[pl.ds(base, L)]
                l0g = l0v[pl.ds(base, L)]
                d0 = e0g * C + l0g
                m0 = l0g < C
                plsc.store_scatter(stv, [d0], tvec, mask=m0)
                plsc.store_scatter(gsv, [d0], g0v[pl.ds(base, L)], mask=m0)
                e1g = e1v[pl.ds(base, L)]
                c0g = plsc.load_gather(cntv, [e1g])
                s1 = l1v[pl.ds(base, L)] + c0g
                d1 = e1g * C + s1
                m1 = s1 < C
                plsc.store_scatter(stv, [d1], tvec, mask=m1)
                plsc.store_scatter(gsv, [d1], g1v[pl.ds(base, L)], mask=m1)
                d0v[pl.ds(base, L)] = jnp.where(m0, d0, dmy16)
                d1v[pl.ds(base, L)] = jnp.where(m1, d1, dmy16)
                return 0
            lax.fori_loop(0, T // L, route, 0)

            pltpu.sync_copy(stv, shst)

            @pl.when(cid == 0)
            def _():
                pltpu.sync_copy(gsv, gs_h)
                pltpu.sync_copy(d0v, d0_h)
                pltpu.sync_copy(d1v, d1_h)

        plsc.subcore_barrier()

        base = wid * SPW
        for j in range(SPW // 64):
            pltpu.sync_copy(shst.at[pl.ds(base + j * 64, 64)], idxv)
            pltpu.async_copy(nm_h.at[idxv], rows, sem).wait()
            pltpu.sync_copy(rows, ei_h.at[pl.ds(base + j * 64, 64)])

    return body(e0, e1, loc0, loc1r, gk0, gk1, cnt0, dmy, normed)


def _combine_sc(xf, ys, d0m, d1m):
    CH = 16

    @functools.partial(
        pl.kernel,
        out_type=jax.ShapeDtypeStruct((T, H), jnp.float32),
        mesh=_MESH,
        scratch_types=[
            pltpu.VMEM((CH,), jnp.int32),
            pltpu.VMEM((CH,), jnp.int32),
            pltpu.VMEM((CH, H), jnp.float32),
            pltpu.VMEM((CH, H), jnp.float32),
            pltpu.VMEM((CH, H), jnp.float32),
            pltpu.SemaphoreType.DMA,
        ],
    )
    def body(x_h, ys_h, d0_h, d1_h, o_h, i0v, i1v, xv, r0v, r1v, sem):
        cid = lax.axis_index("c")
        sid = lax.axis_index("s")
        wid = sid * NC + cid
        for ck in range(TPW // CH):
            tb = wid * TPW + ck * CH
            pltpu.sync_copy(d0_h.at[pl.ds(tb, CH)], i0v)
            pltpu.sync_copy(d1_h.at[pl.ds(tb, CH)], i1v)
            pltpu.sync_copy(x_h.at[pl.ds(tb, CH)], xv)
            cp0 = pltpu.async_copy(ys_h.at[i0v], r0v, sem)
            cp1 = pltpu.async_copy(ys_h.at[i1v], r1v, sem)
            cp0.wait()
            cp1.wait()

            def row(r, _):
                def col(j, _):
                    xv[r, pl.ds(j * L, L)] = (xv[r, pl.ds(j * L, L)]
                                              + r0v[r, pl.ds(j * L, L)]
                                              + r1v[r, pl.ds(j * L, L)])
                    return 0
                lax.fori_loop(0, H // L, col, 0)
                return 0
            lax.fori_loop(0, CH, row, 0)
            pltpu.sync_copy(xv, o_h.at[pl.ds(tb, CH)])

    return body(xf, ys, d0m, d1m)


def kernel(x, gamma, beta, Wg, W1, b1, W2, b2):
    xf = x.reshape(T, H)
    normed, info, stats = _gate_call(xf, gamma.reshape(1, H),
                                     beta.reshape(1, H), Wg)
    e0 = info[:, 0].astype(jnp.int32)
    e1 = info[:, 1].astype(jnp.int32)
    loc0 = info[:, 2].astype(jnp.int32)
    loc1r = info[:, 3].astype(jnp.int32)
    gk0 = info[:, 4]
    gk1 = info[:, 5]
    counts = stats[1]
    l_aux = stats[3, 0]
    cnt0 = jnp.concatenate([stats[0], jnp.zeros((8,), jnp.float32)]
                           ).astype(jnp.int32)
    dmy = jnp.broadcast_to(stats[3, 1], (16,)).astype(jnp.int32)

    expert_in, gk_slot, d0m, d1m = _dispatch_sc(e0, e1, loc0, loc1r,
                                                gk0, gk1, cnt0, dmy, normed)
    ys = _ffn_call(expert_in.reshape(E, C, H), W1, b1, W2, b2,
                   gk_slot.reshape(E, C, 1)).reshape(E * C, H)
    out_flat = _combine_sc(xf, ys, d0m, d1m)
    return out_flat.reshape(B, S, H), l_aux, counts


# speedup vs baseline: 1.5430x; 1.0279x over previous
import functools

import jax
import jax.numpy as jnp
from jax import lax
from jax.experimental import pallas as pl
from jax.experimental.pallas import tpu as pltpu
from jax.experimental.pallas import tpu_sc as plsc

B, S, H, E, K, FF = 1, 2048, 1024, 8, 2, 4096
T = B * S
C = (K * T + E - 1) // E
TB = 128
NB = T // TB
FB = 2048
NF = FF // FB


def _gate_body(x_ref, g_ref, b_ref, wg_ref, normed_ref, info_ref, stats_ref,
               carry):
    i = pl.program_id(0)
    x = x_ref[...]
    mu = jnp.mean(x, axis=-1, keepdims=True)
    xc = x - mu
    var = jnp.mean(xc * xc, axis=-1, keepdims=True)
    normed = xc * lax.rsqrt(var + 1e-5) * g_ref[...] + b_ref[...]
    normed_ref[...] = normed

    logits = jnp.dot(normed, wg_ref[...], preferred_element_type=jnp.float32)
    m = jnp.max(logits, axis=-1, keepdims=True)
    ex = jnp.exp(logits - m)
    gates = ex / jnp.sum(ex, axis=-1, keepdims=True)

    iota = lax.broadcasted_iota(jnp.int32, (TB, E), 1).astype(jnp.float32)
    v0 = jnp.max(gates, axis=-1, keepdims=True)
    e0 = jnp.min(jnp.where(gates == v0, iota, float(E)), axis=-1,
                 keepdims=True)
    mask0 = (iota == e0).astype(jnp.float32)
    g2 = jnp.where(mask0 > 0, -1.0, gates)
    v1 = jnp.max(g2, axis=-1, keepdims=True)
    e1 = jnp.min(jnp.where(g2 == v1, iota, float(E)), axis=-1, keepdims=True)
    mask1 = (iota == e1).astype(jnp.float32)
    denom = jnp.maximum(v0 + v1, 1e-9)
    gk0 = v0 / denom
    gk1 = v1 / denom

    r_io = lax.broadcasted_iota(jnp.int32, (TB, TB), 0)
    c_io = lax.broadcasted_iota(jnp.int32, (TB, TB), 1)
    tri = (c_io < r_io).astype(jnp.float32)
    excl0 = jnp.dot(tri, mask0, preferred_element_type=jnp.float32)
    excl1 = jnp.dot(tri, mask1, preferred_element_type=jnp.float32)

    @pl.when(i == 0)
    def _():
        carry[...] = jnp.zeros_like(carry)

    carry0 = carry[0:1, :]
    carry1 = carry[1:2, :]
    loc0 = jnp.sum((excl0 + carry0) * mask0, axis=-1, keepdims=True)
    loc1 = jnp.sum((excl1 + carry1) * mask1, axis=-1, keepdims=True)
    carry[0:1, :] = carry0 + jnp.sum(mask0, axis=0, keepdims=True)
    carry[1:2, :] = carry1 + jnp.sum(mask1, axis=0, keepdims=True)
    carry[2:3, :] = (jnp.where(i == 0, 0.0, carry[2:3, :])
                     + jnp.sum(gates, axis=0, keepdims=True))

    sel = lambda j: (iota == float(j)).astype(jnp.float32)
    info_ref[...] = (e0 * sel(0) + e1 * sel(1) + loc0 * sel(2)
                     + loc1 * sel(3) + gk0 * sel(4) + gk1 * sel(5))

    @pl.when(i == NB - 1)
    def _():
        count0 = carry[0:1, :]
        count1 = carry[1:2, :]
        sumg = carry[2:3, :]
        total = count0 + count1
        l_aux = (float(E) / (T * T)) * jnp.sum(sumg * count0)
        io8 = lax.broadcasted_iota(jnp.int32, (1, E), 1).astype(jnp.float32)
        mn = jnp.min(total)
        estar = jnp.min(jnp.where(total == mn, io8, float(E)))
        dummyf = estar * C + (C - 1)
        r_io8 = lax.broadcasted_iota(jnp.int32, (E, E), 0)
        c_io8 = lax.broadcasted_iota(jnp.int32, (E, E), 1)
        stats = (jnp.where(r_io8 == 0, jnp.broadcast_to(count0, (E, E)), 0.0)
                 + jnp.where(r_io8 == 1, jnp.broadcast_to(total, (E, E)), 0.0)
                 + jnp.where((r_io8 == 3) & (c_io8 == 0), l_aux, 0.0)
                 + jnp.where((r_io8 == 3) & (c_io8 == 1), dummyf, 0.0))
        stats_ref[...] = stats


def _gate_call(xf, gamma, beta, Wg):
    return pl.pallas_call(
        _gate_body,
        grid=(NB,),
        in_specs=[
            pl.BlockSpec((TB, H), lambda i: (i, 0)),
            pl.BlockSpec((1, H), lambda i: (0, 0)),
            pl.BlockSpec((1, H), lambda i: (0, 0)),
            pl.BlockSpec((H, E), lambda i: (0, 0)),
        ],
        out_specs=[
            pl.BlockSpec((TB, H), lambda i: (i, 0)),
            pl.BlockSpec((TB, E), lambda i: (i, 0)),
            pl.BlockSpec((E, E), lambda i: (0, 0)),
        ],
        out_shape=[
            jax.ShapeDtypeStruct((T, H), jnp.float32),
            jax.ShapeDtypeStruct((T, E), jnp.float32),
            jax.ShapeDtypeStruct((E, E), jnp.float32),
        ],
        scratch_shapes=[pltpu.VMEM((E, E), jnp.float32)],
        compiler_params=pltpu.CompilerParams(
            dimension_semantics=("arbitrary",)),
    )(xf, gamma, beta, Wg)


def _ffn_body(x_ref, w1_ref, b1_ref, w2_ref, b2_ref, gks_ref, y_ref):
    f = pl.program_id(1)
    x = x_ref[0]
    h = jnp.dot(x, w1_ref[0], preferred_element_type=jnp.float32) + b1_ref[0]
    h = 0.5 * h * (1.0 + lax.erf(h * 0.7071067811865476))
    contrib = jnp.dot(h, w2_ref[0], preferred_element_type=jnp.float32)

    @pl.when(f == 0)
    def _():
        y_ref[0] = contrib

    @pl.when(f > 0)
    def _():
        y_ref[0] = y_ref[0] + contrib

    @pl.when(f == NF - 1)
    def _():
        y_ref[0] = (y_ref[0] + b2_ref[0]) * gks_ref[0]


def _ffn_call(expert_in, W1, b1, W2, b2, gks):
    return pl.pallas_call(
        _ffn_body,
        grid=(E, NF),
        in_specs=[
            pl.BlockSpec((1, C, H), lambda e, f: (e, 0, 0)),
            pl.BlockSpec((1, H, FB), lambda e, f: (e, 0, f)),
            pl.BlockSpec((1, 1, FB), lambda e, f: (e * NF + f, 0, 0)),
            pl.BlockSpec((1, FB, H), lambda e, f: (e, f, 0)),
            pl.BlockSpec((1, 1, H), lambda e, f: (e, 0, 0)),
            pl.BlockSpec((1, C, 1), lambda e, f: (e, 0, 0)),
        ],
        out_specs=pl.BlockSpec((1, C, H), lambda e, f: (e, 0, 0)),
        out_shape=jax.ShapeDtypeStruct((E, C, H), jnp.float32),
        compiler_params=pltpu.CompilerParams(
            dimension_semantics=("parallel", "arbitrary")),
    )(expert_in, W1, b1.reshape(E * NF, 1, FB), W2, b2.reshape(E, 1, H), gks)


NC, NS, L = 2, 16, 16
NW = NC * NS
TPW = T // NW
SPW = (E * C) // NW
_MESH = plsc.VectorSubcoreMesh(core_axis_name="c", subcore_axis_name="s",
                               num_cores=NC, num_subcores=NS)


def _dispatch_sc(e0, e1, loc0, loc1r, gk0, gk1, cnt0, dmy, normed):
    @functools.partial(
        pl.kernel,
        out_type=[
            jax.ShapeDtypeStruct((E * C, H), jnp.float32),
            jax.ShapeDtypeStruct((E * C,), jnp.float32),
            jax.ShapeDtypeStruct((T,), jnp.int32),
            jax.ShapeDtypeStruct((T,), jnp.int32),
        ],
        mesh=_MESH,
        scratch_types=[
            pltpu.VMEM((T,), jnp.int32),
            pltpu.VMEM((T,), jnp.int32),
            pltpu.VMEM((T,), jnp.int32),
            pltpu.VMEM((T,), jnp.int32),
            pltpu.VMEM((T,), jnp.float32),
            pltpu.VMEM((T,), jnp.float32),
            pltpu.VMEM((16,), jnp.int32),
            pltpu.VMEM((16,), jnp.int32),
            pltpu.VMEM((E * C,), jnp.int32),
            pltpu.VMEM((E * C,), jnp.float32),
            pltpu.VMEM((T,), jnp.int32),
            pltpu.VMEM((T,), jnp.int32),
            pltpu.VMEM_SHARED((E * C,), jnp.int32),
            pltpu.VMEM((64,), jnp.int32),
            pltpu.VMEM((64, H), jnp.float32),
            pltpu.SemaphoreType.DMA,
        ],
        compiler_params=pltpu.CompilerParams(needs_layout_passes=False),
    )
    def body(e0_h, e1_h, l0_h, l1_h, g0_h, g1_h, c0_h, dm_h, nm_h,
             ei_h, gs_h, d0_h, d1_h,
             e0v, e1v, l0v, l1v, g0v, g1v, cntv, dmyv, stv, gsv, d0v, d1v,
             shst, idxv, rows, sem):
        cid = lax.axis_index("c")
        sid = lax.axis_index("s")
        wid = sid * NC + cid

        @pl.when(sid == 0)
        def _phase1():
            pltpu.sync_copy(e0_h, e0v)
            pltpu.sync_copy(e1_h, e1v)
            pltpu.sync_copy(l0_h, l0v)
            pltpu.sync_copy(l1_h, l1v)
            pltpu.sync_copy(g0_h, g0v)
            pltpu.sync_copy(g1_h, g1v)
            pltpu.sync_copy(c0_h, cntv)
            pltpu.sync_copy(dm_h, dmyv)

            def zinit(j, _):
                stv[pl.ds(j * L, L)] = jnp.zeros((L,), jnp.int32)
                gsv[pl.ds(j * L, L)] = jnp.zeros((L,), jnp.float32)
                return 0
            lax.fori_loop(0, (E * C) // L, zinit, 0)

            dmy16 = dmyv[...]

            def route(g, _):
                base = g * L
                tvec = lax.iota(jnp.int32, L) + base
                e0g = e0v[pl.ds(base, L)]
                l0g = l0v[pl.ds(base, L)]
                d0 = e0g * C + l0g
                m0 = l0g < C
                plsc.store_scatter(stv, [d0], tvec, mask=m0)
                plsc.store_scatter(gsv, [d0], g0v[pl.ds(base, L)], mask=m0)
                e1g = e1v[pl.ds(base, L)]
                c0g = plsc.load_gather(cntv, [e1g])
                s1 = l1v[pl.ds(base, L)] + c0g
                d1 = e1g * C + s1
                m1 = s1 < C
                plsc.store_scatter(stv, [d1], tvec, mask=m1)
                plsc.store_scatter(gsv, [d1], g1v[pl.ds(base, L)], mask=m1)
                d0v[pl.ds(base, L)] = jnp.where(m0, d0, dmy16)
                d1v[pl.ds(base, L)] = jnp.where(m1, d1, dmy16)
                return 0
            lax.fori_loop(0, T // L, route, 0)

            pltpu.sync_copy(stv, shst)

            @pl.when(cid == 0)
            def _():
                pltpu.sync_copy(gsv, gs_h)
                pltpu.sync_copy(d0v, d0_h)
                pltpu.sync_copy(d1v, d1_h)

        plsc.subcore_barrier()

        base = wid * SPW
        for j in range(SPW // 64):
            pltpu.sync_copy(shst.at[pl.ds(base + j * 64, 64)], idxv)
            pltpu.async_copy(nm_h.at[idxv], rows, sem).wait()
            pltpu.sync_copy(rows, ei_h.at[pl.ds(base + j * 64, 64)])

    return body(e0, e1, loc0, loc1r, gk0, gk1, cnt0, dmy, normed)


def _combine_sc(xf, ys, d0m, d1m):
    CH = 16

    @functools.partial(
        pl.kernel,
        out_type=jax.ShapeDtypeStruct((T, H), jnp.float32),
        mesh=_MESH,
        scratch_types=[
            pltpu.VMEM((CH,), jnp.int32),
            pltpu.VMEM((CH,), jnp.int32),
            pltpu.VMEM((CH, H), jnp.float32),
            pltpu.VMEM((CH, H), jnp.float32),
            pltpu.VMEM((CH, H), jnp.float32),
            pltpu.SemaphoreType.DMA,
        ],
    )
    def body(x_h, ys_h, d0_h, d1_h, o_h, i0v, i1v, xv, r0v, r1v, sem):
        cid = lax.axis_index("c")
        sid = lax.axis_index("s")
        wid = sid * NC + cid
        for ck in range(TPW // CH):
            tb = wid * TPW + ck * CH
            pltpu.sync_copy(d0_h.at[pl.ds(tb, CH)], i0v)
            pltpu.sync_copy(d1_h.at[pl.ds(tb, CH)], i1v)
            pltpu.sync_copy(x_h.at[pl.ds(tb, CH)], xv)
            cp0 = pltpu.async_copy(ys_h.at[i0v], r0v, sem)
            cp1 = pltpu.async_copy(ys_h.at[i1v], r1v, sem)
            cp0.wait()
            cp1.wait()

            def row(r, _):
                def col(j, _):
                    xv[r, pl.ds(j * L, L)] = (xv[r, pl.ds(j * L, L)]
                                              + r0v[r, pl.ds(j * L, L)]
                                              + r1v[r, pl.ds(j * L, L)])
                    return 0
                lax.fori_loop(0, H // L, col, 0)
                return 0
            lax.fori_loop(0, CH, row, 0)
            pltpu.sync_copy(xv, o_h.at[pl.ds(tb, CH)])

    return body(xf, ys, d0m, d1m)


def kernel(x, gamma, beta, Wg, W1, b1, W2, b2):
    xf = x.reshape(T, H)
    normed, info, stats = _gate_call(xf, gamma.reshape(1, H),
                                     beta.reshape(1, H), Wg)
    e0 = info[:, 0].astype(jnp.int32)
    e1 = info[:, 1].astype(jnp.int32)
    loc0 = info[:, 2].astype(jnp.int32)
    loc1r = info[:, 3].astype(jnp.int32)
    gk0 = info[:, 4]
    gk1 = info[:, 5]
    counts = stats[1]
    l_aux = stats[3, 0]
    cnt0 = jnp.concatenate([stats[0], jnp.zeros((8,), jnp.float32)]
                           ).astype(jnp.int32)
    dmy = jnp.broadcast_to(stats[3, 1], (16,)).astype(jnp.int32)

    expert_in, gk_slot, d0m, d1m = _dispatch_sc(e0, e1, loc0, loc1r,
                                                gk0, gk1, cnt0, dmy, normed)
    ys = _ffn_call(expert_in.reshape(E, C, H), W1, b1, W2, b2,
                   gk_slot.reshape(E, C, 1)).reshape(E * C, H)
    out_flat = _combine_sc(xf, ys, d0m, d1m)
    return out_flat.reshape(B, S, H), l_aux, counts


# speedup vs baseline: 1.7024x; 1.1034x over previous
import functools

import jax
import jax.numpy as jnp
from jax import lax
from jax.experimental import pallas as pl
from jax.experimental.pallas import tpu as pltpu
from jax.experimental.pallas import tpu_sc as plsc

B, S, H, E, K, FF = 1, 2048, 1024, 8, 2, 4096
T = B * S
C = (K * T + E - 1) // E
TB = 128
NB = T // TB
FB = 2048
NF = FF // FB


def _gate_body(x_ref, g_ref, b_ref, wg_ref, normed_ref, e0_ref, e1_ref,
               l0_ref, l1_ref, g0_ref, g1_ref, cnt_ref, dmy_ref, counts_ref,
               laux_ref, carry):
    i = pl.program_id(0)
    x = x_ref[...]
    mu = jnp.mean(x, axis=-1, keepdims=True)
    xc = x - mu
    var = jnp.mean(xc * xc, axis=-1, keepdims=True)
    normed = xc * lax.rsqrt(var + 1e-5) * g_ref[...] + b_ref[...]
    normed_ref[...] = normed

    logits = jnp.dot(normed, wg_ref[...], preferred_element_type=jnp.float32)
    m = jnp.max(logits, axis=-1, keepdims=True)
    ex = jnp.exp(logits - m)
    gates = ex / jnp.sum(ex, axis=-1, keepdims=True)

    iota = lax.broadcasted_iota(jnp.int32, (TB, E), 1).astype(jnp.float32)
    v0 = jnp.max(gates, axis=-1, keepdims=True)
    e0 = jnp.min(jnp.where(gates == v0, iota, float(E)), axis=-1,
                 keepdims=True)
    mask0 = (iota == e0).astype(jnp.float32)
    g2 = jnp.where(mask0 > 0, -1.0, gates)
    v1 = jnp.max(g2, axis=-1, keepdims=True)
    e1 = jnp.min(jnp.where(g2 == v1, iota, float(E)), axis=-1, keepdims=True)
    mask1 = (iota == e1).astype(jnp.float32)
    denom = jnp.maximum(v0 + v1, 1e-9)
    gk0 = v0 / denom
    gk1 = v1 / denom

    r_io = lax.broadcasted_iota(jnp.int32, (TB, TB), 0)
    c_io = lax.broadcasted_iota(jnp.int32, (TB, TB), 1)
    tri = (c_io < r_io).astype(jnp.float32)
    excl0 = jnp.dot(tri, mask0, preferred_element_type=jnp.float32)
    excl1 = jnp.dot(tri, mask1, preferred_element_type=jnp.float32)

    @pl.when(i == 0)
    def _():
        carry[...] = jnp.zeros_like(carry)

    carry0 = carry[0:1, :]
    carry1 = carry[1:2, :]
    loc0 = jnp.sum((excl0 + carry0) * mask0, axis=-1, keepdims=True)
    loc1 = jnp.sum((excl1 + carry1) * mask1, axis=-1, keepdims=True)
    carry[0:1, :] = carry0 + jnp.sum(mask0, axis=0, keepdims=True)
    carry[1:2, :] = carry1 + jnp.sum(mask1, axis=0, keepdims=True)
    carry[2:3, :] = (jnp.where(i == 0, 0.0, carry[2:3, :])
                     + jnp.sum(gates, axis=0, keepdims=True))

    e0_ref[...] = e0.astype(jnp.int32)
    e1_ref[...] = e1.astype(jnp.int32)
    l0_ref[...] = loc0.astype(jnp.int32)
    l1_ref[...] = loc1.astype(jnp.int32)
    g0_ref[...] = gk0
    g1_ref[...] = gk1

    @pl.when(i == NB - 1)
    def _():
        count0 = carry[0:1, :]
        count1 = carry[1:2, :]
        sumg = carry[2:3, :]
        total = count0 + count1
        l_aux = (float(E) / (T * T)) * jnp.sum(sumg * count0)
        io8 = lax.broadcasted_iota(jnp.int32, (1, E), 1).astype(jnp.float32)
        mn = jnp.min(total)
        estar = jnp.min(jnp.where(total == mn, io8, float(E)))
        dummyf = estar * C + (C - 1)
        cnt_ref[...] = jnp.concatenate(
            [count0, jnp.zeros((1, E), jnp.float32)], axis=1).astype(jnp.int32)
        dmy_ref[...] = jnp.full((1, 16), dummyf).astype(jnp.int32)
        counts_ref[...] = total
        laux_ref[...] = jnp.full((1, 1), l_aux)


def _gate_call(xf, gamma, beta, Wg):
    return pl.pallas_call(
        _gate_body,
        grid=(NB,),
        in_specs=[
            pl.BlockSpec((TB, H), lambda i: (i, 0)),
            pl.BlockSpec((1, H), lambda i: (0, 0)),
            pl.BlockSpec((1, H), lambda i: (0, 0)),
            pl.BlockSpec((H, E), lambda i: (0, 0)),
        ],
        out_specs=[
            pl.BlockSpec((TB, H), lambda i: (i, 0)),
            pl.BlockSpec((TB, 1), lambda i: (i, 0)),
            pl.BlockSpec((TB, 1), lambda i: (i, 0)),
            pl.BlockSpec((TB, 1), lambda i: (i, 0)),
            pl.BlockSpec((TB, 1), lambda i: (i, 0)),
            pl.BlockSpec((TB, 1), lambda i: (i, 0)),
            pl.BlockSpec((TB, 1), lambda i: (i, 0)),
            pl.BlockSpec((1, 16), lambda i: (0, 0)),
            pl.BlockSpec((1, 16), lambda i: (0, 0)),
            pl.BlockSpec((1, E), lambda i: (0, 0)),
            pl.BlockSpec((1, 1), lambda i: (0, 0)),
        ],
        out_shape=[
            jax.ShapeDtypeStruct((T, H), jnp.float32),
            jax.ShapeDtypeStruct((T, 1), jnp.int32),
            jax.ShapeDtypeStruct((T, 1), jnp.int32),
            jax.ShapeDtypeStruct((T, 1), jnp.int32),
            jax.ShapeDtypeStruct((T, 1), jnp.int32),
            jax.ShapeDtypeStruct((T, 1), jnp.float32),
            jax.ShapeDtypeStruct((T, 1), jnp.float32),
            jax.ShapeDtypeStruct((1, 16), jnp.int32),
            jax.ShapeDtypeStruct((1, 16), jnp.int32),
            jax.ShapeDtypeStruct((1, E), jnp.float32),
            jax.ShapeDtypeStruct((1, 1), jnp.float32),
        ],
        scratch_shapes=[pltpu.VMEM((E, E), jnp.float32)],
        compiler_params=pltpu.CompilerParams(
            dimension_semantics=("arbitrary",)),
    )(xf, gamma, beta, Wg)


def _ffn_body(x_ref, w1_ref, b1_ref, w2_ref, b2_ref, gks0_ref, gks1_ref,
              y_ref):
    f = pl.program_id(1)
    x = x_ref[0]
    h = jnp.dot(x, w1_ref[0], preferred_element_type=jnp.float32) + b1_ref[0]
    h = 0.5 * h * (1.0 + lax.erf(h * 0.7071067811865476))
    contrib = jnp.dot(h, w2_ref[0], preferred_element_type=jnp.float32)

    @pl.when(f == 0)
    def _():
        y_ref[0] = contrib

    @pl.when(f > 0)
    def _():
        y_ref[0] = y_ref[0] + contrib

    @pl.when(f == NF - 1)
    def _():
        y_ref[0] = (y_ref[0] + b2_ref[0]) * (gks0_ref[0] + gks1_ref[0])


def _ffn_call(expert_in, W1, b1, W2, b2, gks0, gks1):
    return pl.pallas_call(
        _ffn_body,
        grid=(E, NF),
        in_specs=[
            pl.BlockSpec((1, C, H), lambda e, f: (e, 0, 0)),
            pl.BlockSpec((1, H, FB), lambda e, f: (e, 0, f)),
            pl.BlockSpec((1, 1, FB), lambda e, f: (e * NF + f, 0, 0)),
            pl.BlockSpec((1, FB, H), lambda e, f: (e, f, 0)),
            pl.BlockSpec((1, 1, H), lambda e, f: (e, 0, 0)),
            pl.BlockSpec((1, C, 1), lambda e, f: (e, 0, 0)),
            pl.BlockSpec((1, C, 1), lambda e, f: (e, 0, 0)),
        ],
        out_specs=pl.BlockSpec((1, C, H), lambda e, f: (e, 0, 0)),
        out_shape=jax.ShapeDtypeStruct((E, C, H), jnp.float32),
        compiler_params=pltpu.CompilerParams(
            dimension_semantics=("parallel", "arbitrary")),
    )(expert_in, W1, b1.reshape(E * NF, 1, FB), W2, b2.reshape(E, 1, H),
      gks0, gks1)


NC, NS, L = 2, 16, 16
NW = NC * NS
TPW = T // NW
SPW = (E * C) // NW
_MESH = plsc.VectorSubcoreMesh(core_axis_name="c", subcore_axis_name="s",
                               num_cores=NC, num_subcores=NS)


GCH = 32


def _dispatch_sc(e0, e1, loc0, loc1r, gk0, gk1, cnt0, dmy, normed):
    @functools.partial(
        pl.kernel,
        out_type=[
            jax.ShapeDtypeStruct((E * C, H), jnp.float32),
            jax.ShapeDtypeStruct((E * C,), jnp.float32),
            jax.ShapeDtypeStruct((E * C,), jnp.float32),
            jax.ShapeDtypeStruct((T,), jnp.int32),
            jax.ShapeDtypeStruct((T,), jnp.int32),
        ],
        mesh=_MESH,
        scratch_types=[
            pltpu.VMEM((T,), jnp.int32),
            pltpu.VMEM((T,), jnp.int32),
            pltpu.VMEM((T,), jnp.float32),
            pltpu.VMEM((16,), jnp.int32),
            pltpu.VMEM((16,), jnp.int32),
            pltpu.VMEM((E * C,), jnp.int32),
            pltpu.VMEM((E * C,), jnp.float32),
            pltpu.VMEM((T,), jnp.int32),
            pltpu.VMEM_SHARED((E * C,), jnp.int32),
            pltpu.VMEM_SHARED((E * C,), jnp.int32),
            pltpu.VMEM((SPW,), jnp.int32),
            pltpu.VMEM((SPW,), jnp.int32),
            pltpu.VMEM((SPW,), jnp.int32),
            pltpu.VMEM((GCH, H), jnp.float32),
            pltpu.VMEM((GCH, H), jnp.float32),
            pltpu.SemaphoreType.DMA,
            pltpu.SemaphoreType.DMA,
            pltpu.SemaphoreType.DMA,
            pltpu.SemaphoreType.DMA,
        ],
        compiler_params=pltpu.CompilerParams(needs_layout_passes=False),
    )
    def body(e0_h, e1_h, l0_h, l1_h, g0_h, g1_h, c0_h, dm_h, nm_h,
             ei_h, gs0_h, gs1_h, d0_h, d1_h,
             evx, locv, gkv, cntv, dmyv, stv, gsv, dxv, shst0, shst1,
             st0v, st1v, idxv, rowsA, rowsB, semA, semB, semoA, semoB):
        cid = lax.axis_index("c")
        sid = lax.axis_index("s")
        wid = sid * NC + cid

        def phase1(e_h, l_h, g_h, with_cnt, sh_ref, gk_hbm, dx_hbm):
            pltpu.sync_copy(e_h, evx)
            pltpu.sync_copy(l_h, locv)
            pltpu.sync_copy(g_h, gkv)
            pltpu.sync_copy(dm_h, dmyv)
            if with_cnt:
                pltpu.sync_copy(c0_h, cntv)

            def zinit(j, _):
                stv[pl.ds(j * L, L)] = jnp.zeros((L,), jnp.int32)
                gsv[pl.ds(j * L, L)] = jnp.zeros((L,), jnp.float32)
                return 0
            lax.fori_loop(0, (E * C) // L, zinit, 0)

            dmy16 = dmyv[...]

            def route(g, _):
                base = g * L
                tvec = lax.iota(jnp.int32, L) + base
                eg = evx[pl.ds(base, L)]
                lg = locv[pl.ds(base, L)]
                if with_cnt:
                    lg = lg + plsc.load_gather(cntv, [eg])
                d = eg * C + lg
                m = lg < C
                plsc.store_scatter(stv, [d], tvec, mask=m)
                plsc.store_scatter(gsv, [d], gkv[pl.ds(base, L)], mask=m)
                dxv[pl.ds(base, L)] = jnp.where(m, d, dmy16)
                return 0
            lax.fori_loop(0, T // L, route, 0)

            pltpu.sync_copy(stv, sh_ref)

            @pl.when(cid == 0)
            def _():
                pltpu.sync_copy(gsv, gk_hbm)
                pltpu.sync_copy(dxv, dx_hbm)

        @pl.when(sid == 0)
        def _():
            phase1(e0_h, l0_h, g0_h, False, shst0, gs0_h, d0_h)

        @pl.when(sid == 1)
        def _():
            phase1(e1_h, l1_h, g1_h, True, shst1, gs1_h, d1_h)

        plsc.subcore_barrier()

        base = wid * SPW
        pltpu.sync_copy(shst0.at[pl.ds(base, SPW)], st0v)
        pltpu.sync_copy(shst1.at[pl.ds(base, SPW)], st1v)

        def merge(j, _):
            idxv[pl.ds(j * L, L)] = (st0v[pl.ds(j * L, L)]
                                     + st1v[pl.ds(j * L, L)])
            return 0
        lax.fori_loop(0, SPW // L, merge, 0)

        nck = SPW // GCH
        sets = [(rowsA, semA, semoA), (rowsB, semB, semoB)]
        incps, outcps = {}, {}

        def fire_in(ck):
            rows, sem, _ = sets[ck % 2]
            return pltpu.async_copy(nm_h.at[idxv.at[pl.ds(ck * GCH, GCH)]],
                                    rows, sem)

        incps[0] = fire_in(0)
        for ck in range(nck):
            rows, sem, semo = sets[ck % 2]
            incps[ck].wait()
            if ck + 1 < nck:
                if ck - 1 >= 0:
                    outcps[ck - 1].wait()
                incps[ck + 1] = fire_in(ck + 1)
            outcps[ck] = pltpu.async_copy(
                rows, ei_h.at[pl.ds(base + ck * GCH, GCH)], semo)
        for ck in range(max(0, nck - 2), nck):
            outcps[ck].wait()

    return body(e0, e1, loc0, loc1r, gk0, gk1, cnt0, dmy, normed)


def _combine_sc(xf, ys, d0m, d1m):
    CH = 8

    @functools.partial(
        pl.kernel,
        out_type=jax.ShapeDtypeStruct((T, H), jnp.float32),
        mesh=_MESH,
        scratch_types=(
            [pltpu.VMEM((TPW,), jnp.int32)] * 2
            + [pltpu.VMEM((CH, H), jnp.float32)] * 9
            + [pltpu.SemaphoreType.DMA] * 6
        ),
        compiler_params=pltpu.CompilerParams(needs_layout_passes=False),
    )
    def body(x_h, ys_h, d0_h, d1_h, o_h, i0v, i1v,
             xv0, r00, r10, xv1, r01, r11, xv2, r02, r12,
             sin0, sin1, sin2, sout0, sout1, sout2):
        cid = lax.axis_index("c")
        sid = lax.axis_index("s")
        wid = sid * NC + cid
        base = wid * TPW
        pltpu.sync_copy(d0_h.at[pl.ds(base, TPW)], i0v)
        pltpu.sync_copy(d1_h.at[pl.ds(base, TPW)], i1v)

        sets = [(xv0, r00, r10, sin0, sout0),
                (xv1, r01, r11, sin1, sout1),
                (xv2, r02, r12, sin2, sout2)]
        nck = TPW // CH
        incps, outcps = {}, {}

        def fire_in(ck):
            xv, r0, r1, sin, _ = sets[ck % 3]
            tb = base + ck * CH
            return [
                pltpu.async_copy(x_h.at[pl.ds(tb, CH)], xv, sin),
                pltpu.async_copy(ys_h.at[i0v.at[pl.ds(ck * CH, CH)]], r0,
                                 sin),
                pltpu.async_copy(ys_h.at[i1v.at[pl.ds(ck * CH, CH)]], r1,
                                 sin),
            ]

        incps[0] = fire_in(0)
        incps[1] = fire_in(1)
        for ck in range(nck):
            xv, r0, r1, sin, sout = sets[ck % 3]
            for cp in incps[ck]:
                cp.wait()

            def col(j, _):
                for r in range(CH):
                    xv[r, pl.ds(j * L, L)] = (xv[r, pl.ds(j * L, L)]
                                              + r0[r, pl.ds(j * L, L)]
                                              + r1[r, pl.ds(j * L, L)])
                return 0
            lax.fori_loop(0, H // L, col, 0)
            outcps[ck] = pltpu.async_copy(
                xv, o_h.at[pl.ds(base + ck * CH, CH)], sout)
            if ck + 2 < nck:
                if ck - 1 >= 0:
                    outcps[ck - 1].wait()
                incps[ck + 2] = fire_in(ck + 2)
        for ck in range(max(0, nck - 3), nck):
            outcps[ck].wait()

    return body(xf, ys, d0m, d1m)


def kernel(x, gamma, beta, Wg, W1, b1, W2, b2):
    xf = x.reshape(T, H)
    (normed, e0, e1, loc0, loc1r, gk0, gk1, cnt0, dmy, counts2d,
     laux2d) = _gate_call(xf, gamma.reshape(1, H), beta.reshape(1, H), Wg)
    counts = counts2d.reshape(E)
    l_aux = laux2d.reshape(())

    expert_in, gks0, gks1, d0m, d1m = _dispatch_sc(
        e0.reshape(T), e1.reshape(T), loc0.reshape(T), loc1r.reshape(T),
        gk0.reshape(T), gk1.reshape(T), cnt0.reshape(16), dmy.reshape(16),
        normed)
    ys = _ffn_call(expert_in.reshape(E, C, H), W1, b1, W2, b2,
                   gks0.reshape(E, C, 1), gks1.reshape(E, C, 1)
                   ).reshape(E * C, H)
    out_flat = _combine_sc(xf, ys, d0m, d1m)
    return out_flat.reshape(B, S, H), l_aux, counts


# speedup vs baseline: 1.8896x; 1.1099x over previous
import functools

import jax
import jax.numpy as jnp
from jax import lax
from jax.experimental import pallas as pl
from jax.experimental.pallas import tpu as pltpu
from jax.experimental.pallas import tpu_sc as plsc

B, S, H, E, K, FF = 1, 2048, 1024, 8, 2, 4096
T = B * S
C = (K * T + E - 1) // E
TB = 128
NB = T // TB
FB = 2048
NF = FF // FB


def _gate_body(x_ref, g_ref, b_ref, wg_ref, normed_ref, rt_ref, cnt_ref,
               dmy_ref, counts_ref, laux_ref, carry):
    i = pl.program_id(0)
    x = x_ref[...]
    mu = jnp.mean(x, axis=-1, keepdims=True)
    xc = x - mu
    var = jnp.mean(xc * xc, axis=-1, keepdims=True)
    normed = xc * lax.rsqrt(var + 1e-5) * g_ref[...] + b_ref[...]
    normed_ref[...] = normed

    logits = jnp.dot(normed, wg_ref[...], preferred_element_type=jnp.float32)
    m = jnp.max(logits, axis=-1, keepdims=True)
    ex = jnp.exp(logits - m)
    gates = ex / jnp.sum(ex, axis=-1, keepdims=True)

    iota = lax.broadcasted_iota(jnp.int32, (TB, E), 1).astype(jnp.float32)
    v0 = jnp.max(gates, axis=-1, keepdims=True)
    e0 = jnp.min(jnp.where(gates == v0, iota, float(E)), axis=-1,
                 keepdims=True)
    mask0 = (iota == e0).astype(jnp.float32)
    g2 = jnp.where(mask0 > 0, -1.0, gates)
    v1 = jnp.max(g2, axis=-1, keepdims=True)
    e1 = jnp.min(jnp.where(g2 == v1, iota, float(E)), axis=-1, keepdims=True)
    mask1 = (iota == e1).astype(jnp.float32)
    denom = jnp.maximum(v0 + v1, 1e-9)
    gk0 = v0 / denom
    gk1 = v1 / denom

    r_io = lax.broadcasted_iota(jnp.int32, (TB, TB), 0)
    c_io = lax.broadcasted_iota(jnp.int32, (TB, TB), 1)
    tri = (c_io < r_io).astype(jnp.float32)
    excl0 = jnp.dot(tri, mask0, preferred_element_type=jnp.float32)
    excl1 = jnp.dot(tri, mask1, preferred_element_type=jnp.float32)

    @pl.when(i == 0)
    def _():
        carry[...] = jnp.zeros_like(carry)

    carry0 = carry[0:1, :]
    carry1 = carry[1:2, :]
    loc0 = jnp.sum((excl0 + carry0) * mask0, axis=-1, keepdims=True)
    loc1 = jnp.sum((excl1 + carry1) * mask1, axis=-1, keepdims=True)
    carry[0:1, :] = carry0 + jnp.sum(mask0, axis=0, keepdims=True)
    carry[1:2, :] = carry1 + jnp.sum(mask1, axis=0, keepdims=True)
    carry[2:3, :] = (jnp.where(i == 0, 0.0, carry[2:3, :])
                     + jnp.sum(gates, axis=0, keepdims=True))

    sel = lambda j: (iota == float(j)).astype(jnp.float32)
    info = (e0 * sel(0) + e1 * sel(1) + loc0 * sel(2)
            + loc1 * sel(3) + gk0 * sel(4) + gk1 * sel(5))
    eye = (r_io == c_io).astype(jnp.float32)
    rt_ref[...] = lax.dot_general(info, eye, (((0,), (0,)), ((), ())),
                                  preferred_element_type=jnp.float32)

    @pl.when(i == NB - 1)
    def _():
        count0 = carry[0:1, :]
        count1 = carry[1:2, :]
        sumg = carry[2:3, :]
        total = count0 + count1
        l_aux = (float(E) / (T * T)) * jnp.sum(sumg * count0)
        io8 = lax.broadcasted_iota(jnp.int32, (1, E), 1).astype(jnp.float32)
        mn = jnp.min(total)
        estar = jnp.min(jnp.where(total == mn, io8, float(E)))
        dummyf = estar * C + (C - 1)
        cnt_ref[...] = jnp.concatenate(
            [count0, jnp.zeros((1, E), jnp.float32)], axis=1).astype(jnp.int32)
        dmy_ref[...] = jnp.full((1, 16), dummyf).astype(jnp.int32)
        counts_ref[...] = total
        laux_ref[...] = jnp.full((1, 1), l_aux)


def _gate_call(xf, gamma, beta, Wg):
    return pl.pallas_call(
        _gate_body,
        grid=(NB,),
        in_specs=[
            pl.BlockSpec((TB, H), lambda i: (i, 0)),
            pl.BlockSpec((1, H), lambda i: (0, 0)),
            pl.BlockSpec((1, H), lambda i: (0, 0)),
            pl.BlockSpec((H, E), lambda i: (0, 0)),
        ],
        out_specs=[
            pl.BlockSpec((TB, H), lambda i: (i, 0)),
            pl.BlockSpec((E, TB), lambda i: (0, i)),
            pl.BlockSpec((1, 16), lambda i: (0, 0)),
            pl.BlockSpec((1, 16), lambda i: (0, 0)),
            pl.BlockSpec((1, E), lambda i: (0, 0)),
            pl.BlockSpec((1, 1), lambda i: (0, 0)),
        ],
        out_shape=[
            jax.ShapeDtypeStruct((T, H), jnp.float32),
            jax.ShapeDtypeStruct((E, T), jnp.float32),
            jax.ShapeDtypeStruct((1, 16), jnp.int32),
            jax.ShapeDtypeStruct((1, 16), jnp.int32),
            jax.ShapeDtypeStruct((1, E), jnp.float32),
            jax.ShapeDtypeStruct((1, 1), jnp.float32),
        ],
        scratch_shapes=[pltpu.VMEM((E, E), jnp.float32)],
        compiler_params=pltpu.CompilerParams(
            dimension_semantics=("arbitrary",)),
    )(xf, gamma, beta, Wg)


def _ffn_body(x_ref, w1_ref, b1_ref, w2_ref, b2_ref, y_ref):
    f = pl.program_id(1)
    x = x_ref[0]
    h = jnp.dot(x, w1_ref[0], preferred_element_type=jnp.float32) + b1_ref[0]
    h = 0.5 * h * (1.0 + lax.erf(h * 0.7071067811865476))
    contrib = jnp.dot(h, w2_ref[0], preferred_element_type=jnp.float32)

    @pl.when(f == 0)
    def _():
        y_ref[0] = contrib

    @pl.when(f > 0)
    def _():
        y_ref[0] = y_ref[0] + contrib

    @pl.when(f == NF - 1)
    def _():
        y_ref[0] = y_ref[0] + b2_ref[0]


def _ffn_call(expert_in, W1, b1, W2, b2):
    return pl.pallas_call(
        _ffn_body,
        grid=(E, NF),
        in_specs=[
            pl.BlockSpec((1, C, H), lambda e, f: (e, 0, 0)),
            pl.BlockSpec((1, H, FB), lambda e, f: (e, 0, f)),
            pl.BlockSpec((1, 1, FB), lambda e, f: (e * NF + f, 0, 0)),
            pl.BlockSpec((1, FB, H), lambda e, f: (e, f, 0)),
            pl.BlockSpec((1, 1, H), lambda e, f: (e, 0, 0)),
        ],
        out_specs=pl.BlockSpec((1, C, H), lambda e, f: (e, 0, 0)),
        out_shape=jax.ShapeDtypeStruct((E, C, H), jnp.float32),
        compiler_params=pltpu.CompilerParams(
            dimension_semantics=("parallel", "arbitrary")),
    )(expert_in, W1, b1.reshape(E * NF, 1, FB), W2, b2.reshape(E, 1, H))


NC, NS, L = 2, 16, 16
NW = NC * NS
TPW = T // NW
SPW = (E * C) // NW
_MESH = plsc.VectorSubcoreMesh(core_axis_name="c", subcore_axis_name="s",
                               num_cores=NC, num_subcores=NS)


GCH = 32


def _dispatch_sc(routing, cnt0, dmy, normed):
    @functools.partial(
        pl.kernel,
        out_type=[
            jax.ShapeDtypeStruct((E * C, H), jnp.float32),
            jax.ShapeDtypeStruct((T,), jnp.int32),
            jax.ShapeDtypeStruct((T,), jnp.int32),
            jax.ShapeDtypeStruct((T,), jnp.float32),
            jax.ShapeDtypeStruct((T,), jnp.float32),
        ],
        mesh=_MESH,
        scratch_types=[
            pltpu.VMEM((E, T), jnp.float32),
            pltpu.VMEM((16,), jnp.int32),
            pltpu.VMEM((16,), jnp.int32),
            pltpu.VMEM((E * C,), jnp.int32),
            pltpu.VMEM((T,), jnp.int32),
            pltpu.VMEM((T,), jnp.float32),
            pltpu.VMEM_SHARED((E * C,), jnp.int32),
            pltpu.VMEM_SHARED((E * C,), jnp.int32),
            pltpu.VMEM((SPW,), jnp.int32),
            pltpu.VMEM((SPW,), jnp.int32),
            pltpu.VMEM((SPW,), jnp.int32),
            pltpu.VMEM((GCH, H), jnp.float32),
            pltpu.VMEM((GCH, H), jnp.float32),
            pltpu.SemaphoreType.DMA,
            pltpu.SemaphoreType.DMA,
            pltpu.SemaphoreType.DMA,
            pltpu.SemaphoreType.DMA,
        ],
        compiler_params=pltpu.CompilerParams(needs_layout_passes=False),
    )
    def body(rt_h, c0_h, dm_h, nm_h,
             ei_h, d0_h, d1_h, g0_h, g1_h,
             routv, cntv, dmyv, stv, dxv, gmv, shst0, shst1,
             st0v, st1v, idxv, rowsA, rowsB, semA, semB, semoA, semoB):
        cid = lax.axis_index("c")
        sid = lax.axis_index("s")
        wid = sid * NC + cid

        def phase1(krow, with_cnt, sh_ref, dx_hbm, gk_hbm):
            pltpu.sync_copy(rt_h, routv)
            pltpu.sync_copy(dm_h.at[0], dmyv)
            if with_cnt:
                pltpu.sync_copy(c0_h.at[0], cntv)

            def zinit(j, _):
                stv[pl.ds(j * L, L)] = jnp.zeros((L,), jnp.int32)
                return 0
            lax.fori_loop(0, (E * C) // L, zinit, 0)

            dmy16 = dmyv[...]

            def route(g, _):
                base = g * L
                tvec = lax.iota(jnp.int32, L) + base
                eg = routv[krow, pl.ds(base, L)].astype(jnp.int32)
                lg = routv[krow + 2, pl.ds(base, L)].astype(jnp.int32)
                gk = routv[krow + 4, pl.ds(base, L)]
                if with_cnt:
                    lg = lg + plsc.load_gather(cntv, [eg])
                d = eg * C + lg
                m = lg < C
                plsc.store_scatter(stv, [d], tvec, mask=m)
                dxv[pl.ds(base, L)] = jnp.where(m, d, dmy16)
                gmv[pl.ds(base, L)] = jnp.where(m, gk, 0.0)
                return 0
            lax.fori_loop(0, T // L, route, 0)

            pltpu.sync_copy(stv, sh_ref)

            @pl.when(cid == 0)
            def _():
                pltpu.sync_copy(dxv, dx_hbm)
                pltpu.sync_copy(gmv, gk_hbm)

        @pl.when(sid == 0)
        def _():
            phase1(0, False, shst0, d0_h, g0_h)

        @pl.when(sid == 1)
        def _():
            phase1(1, True, shst1, d1_h, g1_h)

        plsc.subcore_barrier()

        base = wid * SPW
        pltpu.sync_copy(shst0.at[pl.ds(base, SPW)], st0v)
        pltpu.sync_copy(shst1.at[pl.ds(base, SPW)], st1v)

        def merge(j, _):
            idxv[pl.ds(j * L, L)] = (st0v[pl.ds(j * L, L)]
                                     + st1v[pl.ds(j * L, L)])
            return 0
        lax.fori_loop(0, SPW // L, merge, 0)

        nck = SPW // GCH
        sets = [(rowsA, semA, semoA), (rowsB, semB, semoB)]
        incps, outcps = {}, {}

        def fire_in(ck):
            rows, sem, _ = sets[ck % 2]
            return pltpu.async_copy(nm_h.at[idxv.at[pl.ds(ck * GCH, GCH)]],
                                    rows, sem)

        incps[0] = fire_in(0)
        for ck in range(nck):
            rows, sem, semo = sets[ck % 2]
            incps[ck].wait()
            if ck + 1 < nck:
                if ck - 1 >= 0:
                    outcps[ck - 1].wait()
                incps[ck + 1] = fire_in(ck + 1)
            outcps[ck] = pltpu.async_copy(
                rows, ei_h.at[pl.ds(base + ck * GCH, GCH)], semo)
        for ck in range(max(0, nck - 2), nck):
            outcps[ck].wait()

    return body(routing, cnt0, dmy, normed)


def _combine_sc(xf, ys, d0m, d1m, g0m, g1m):
    CH = 8

    @functools.partial(
        pl.kernel,
        out_type=jax.ShapeDtypeStruct((T, H), jnp.float32),
        mesh=_MESH,
        scratch_types=(
            [pltpu.VMEM((TPW,), jnp.int32)] * 2
            + [pltpu.VMEM((TPW,), jnp.float32)] * 2
            + [pltpu.VMEM((CH, H), jnp.float32)] * 9
            + [pltpu.SemaphoreType.DMA] * 6
        ),
        compiler_params=pltpu.CompilerParams(needs_layout_passes=False),
    )
    def body(x_h, ys_h, d0_h, d1_h, g0_h, g1_h, o_h, i0v, i1v, w0v, w1v,
             xv0, r00, r10, xv1, r01, r11, xv2, r02, r12,
             sin0, sin1, sin2, sout0, sout1, sout2):
        cid = lax.axis_index("c")
        sid = lax.axis_index("s")
        wid = sid * NC + cid
        base = wid * TPW
        pltpu.sync_copy(d0_h.at[pl.ds(base, TPW)], i0v)
        pltpu.sync_copy(d1_h.at[pl.ds(base, TPW)], i1v)
        pltpu.sync_copy(g0_h.at[pl.ds(base, TPW)], w0v)
        pltpu.sync_copy(g1_h.at[pl.ds(base, TPW)], w1v)

        sets = [(xv0, r00, r10, sin0, sout0),
                (xv1, r01, r11, sin1, sout1),
                (xv2, r02, r12, sin2, sout2)]
        nck = TPW // CH
        incps, outcps = {}, {}

        def fire_in(ck):
            xv, r0, r1, sin, _ = sets[ck % 3]
            tb = base + ck * CH
            return [
                pltpu.async_copy(x_h.at[pl.ds(tb, CH)], xv, sin),
                pltpu.async_copy(ys_h.at[i0v.at[pl.ds(ck * CH, CH)]], r0,
                                 sin),
                pltpu.async_copy(ys_h.at[i1v.at[pl.ds(ck * CH, CH)]], r1,
                                 sin),
            ]

        incps[0] = fire_in(0)
        incps[1] = fire_in(1)
        for ck in range(nck):
            xv, r0, r1, sin, sout = sets[ck % 3]
            for cp in incps[ck]:
                cp.wait()

            ws = [(plsc.load_gather(w0v, [jnp.full((L,), ck * CH + r,
                                                   jnp.int32)]),
                   plsc.load_gather(w1v, [jnp.full((L,), ck * CH + r,
                                                   jnp.int32)]))
                  for r in range(CH)]

            def col(j, _):
                for r in range(CH):
                    w0, w1 = ws[r]
                    xv[r, pl.ds(j * L, L)] = (xv[r, pl.ds(j * L, L)]
                                              + w0 * r0[r, pl.ds(j * L, L)]
                                              + w1 * r1[r, pl.ds(j * L, L)])
                return 0
            lax.fori_loop(0, H // L, col, 0)
            outcps[ck] = pltpu.async_copy(
                xv, o_h.at[pl.ds(base + ck * CH, CH)], sout)
            if ck + 2 < nck:
                if ck - 1 >= 0:
                    outcps[ck - 1].wait()
                incps[ck + 2] = fire_in(ck + 2)
        for ck in range(max(0, nck - 3), nck):
            outcps[ck].wait()

    return body(xf, ys, d0m, d1m, g0m, g1m)


def kernel(x, gamma, beta, Wg, W1, b1, W2, b2):
    xf = x.reshape(T, H)
    normed, routing, cnt0, dmy, counts2d, laux2d = _gate_call(
        xf, gamma.reshape(1, H), beta.reshape(1, H), Wg)
    counts = counts2d.reshape(E)
    l_aux = laux2d.reshape(())

    expert_in, d0m, d1m, g0m, g1m = _dispatch_sc(routing, cnt0, dmy, normed)
    ys = _ffn_call(expert_in.reshape(E, C, H), W1, b1, W2, b2
                   ).reshape(E * C, H)
    out_flat = _combine_sc(xf, ys, d0m, d1m, g0m, g1m)
    return out_flat.reshape(B, S, H), l_aux, counts


# speedup vs baseline: 1.9043x; 1.0078x over previous
import functools

import jax
import jax.numpy as jnp
from jax import lax
from jax.experimental import pallas as pl
from jax.experimental.pallas import tpu as pltpu
from jax.experimental.pallas import tpu_sc as plsc

B, S, H, E, K, FF = 1, 2048, 1024, 8, 2, 4096
T = B * S
C = (K * T + E - 1) // E
TB = 128
NB = T // TB
FB = 2048
NF = FF // FB


def _gate_body(x_ref, g_ref, b_ref, wg_ref, normed_ref, rt_ref, cnt_ref,
               dmy_ref, counts_ref, laux_ref, carry):
    i = pl.program_id(0)
    x = x_ref[...]
    mu = jnp.mean(x, axis=-1, keepdims=True)
    xc = x - mu
    var = jnp.mean(xc * xc, axis=-1, keepdims=True)
    normed = xc * lax.rsqrt(var + 1e-5) * g_ref[...] + b_ref[...]
    normed_ref[...] = normed

    logits = jnp.dot(normed, wg_ref[...], preferred_element_type=jnp.float32)
    m = jnp.max(logits, axis=-1, keepdims=True)
    ex = jnp.exp(logits - m)
    gates = ex / jnp.sum(ex, axis=-1, keepdims=True)

    iota = lax.broadcasted_iota(jnp.int32, (TB, E), 1).astype(jnp.float32)
    v0 = jnp.max(gates, axis=-1, keepdims=True)
    e0 = jnp.min(jnp.where(gates == v0, iota, float(E)), axis=-1,
                 keepdims=True)
    mask0 = (iota == e0).astype(jnp.float32)
    g2 = jnp.where(mask0 > 0, -1.0, gates)
    v1 = jnp.max(g2, axis=-1, keepdims=True)
    e1 = jnp.min(jnp.where(g2 == v1, iota, float(E)), axis=-1, keepdims=True)
    mask1 = (iota == e1).astype(jnp.float32)
    denom = jnp.maximum(v0 + v1, 1e-9)
    gk0 = v0 / denom
    gk1 = v1 / denom

    r_io = lax.broadcasted_iota(jnp.int32, (TB, TB), 0)
    c_io = lax.broadcasted_iota(jnp.int32, (TB, TB), 1)
    tri = (c_io < r_io).astype(jnp.float32)
    excl0 = jnp.dot(tri, mask0, preferred_element_type=jnp.float32)
    excl1 = jnp.dot(tri, mask1, preferred_element_type=jnp.float32)

    @pl.when(i == 0)
    def _():
        carry[...] = jnp.zeros_like(carry)

    carry0 = carry[0:1, :]
    carry1 = carry[1:2, :]
    loc0 = jnp.sum((excl0 + carry0) * mask0, axis=-1, keepdims=True)
    loc1 = jnp.sum((excl1 + carry1) * mask1, axis=-1, keepdims=True)
    carry[0:1, :] = carry0 + jnp.sum(mask0, axis=0, keepdims=True)
    carry[1:2, :] = carry1 + jnp.sum(mask1, axis=0, keepdims=True)
    carry[2:3, :] = (jnp.where(i == 0, 0.0, carry[2:3, :])
                     + jnp.sum(gates, axis=0, keepdims=True))

    sel = lambda j: (iota == float(j)).astype(jnp.float32)
    info = (e0 * sel(0) + e1 * sel(1) + loc0 * sel(2)
            + loc1 * sel(3) + gk0 * sel(4) + gk1 * sel(5))
    rt_ref[...] = jnp.transpose(info, (1, 0))

    @pl.when(i == NB - 1)
    def _():
        count0 = carry[0:1, :]
        count1 = carry[1:2, :]
        sumg = carry[2:3, :]
        total = count0 + count1
        l_aux = (float(E) / (T * T)) * jnp.sum(sumg * count0)
        io8 = lax.broadcasted_iota(jnp.int32, (1, E), 1).astype(jnp.float32)
        mn = jnp.min(total)
        estar = jnp.min(jnp.where(total == mn, io8, float(E)))
        dummyf = estar * C + (C - 1)
        cnt_ref[...] = jnp.concatenate(
            [count0, jnp.zeros((1, E), jnp.float32)], axis=1).astype(jnp.int32)
        dmy_ref[...] = jnp.full((1, 16), dummyf).astype(jnp.int32)
        counts_ref[...] = total
        laux_ref[...] = jnp.full((1, 1), l_aux)


def _gate_call(xf, gamma, beta, Wg):
    return pl.pallas_call(
        _gate_body,
        grid=(NB,),
        in_specs=[
            pl.BlockSpec((TB, H), lambda i: (i, 0)),
            pl.BlockSpec((1, H), lambda i: (0, 0)),
            pl.BlockSpec((1, H), lambda i: (0, 0)),
            pl.BlockSpec((H, E), lambda i: (0, 0)),
        ],
        out_specs=[
            pl.BlockSpec((TB, H), lambda i: (i, 0)),
            pl.BlockSpec((E, TB), lambda i: (0, i)),
            pl.BlockSpec((1, 16), lambda i: (0, 0)),
            pl.BlockSpec((1, 16), lambda i: (0, 0)),
            pl.BlockSpec((1, E), lambda i: (0, 0)),
            pl.BlockSpec((1, 1), lambda i: (0, 0)),
        ],
        out_shape=[
            jax.ShapeDtypeStruct((T, H), jnp.float32),
            jax.ShapeDtypeStruct((E, T), jnp.float32),
            jax.ShapeDtypeStruct((1, 16), jnp.int32),
            jax.ShapeDtypeStruct((1, 16), jnp.int32),
            jax.ShapeDtypeStruct((1, E), jnp.float32),
            jax.ShapeDtypeStruct((1, 1), jnp.float32),
        ],
        scratch_shapes=[pltpu.VMEM((E, E), jnp.float32)],
        compiler_params=pltpu.CompilerParams(
            dimension_semantics=("arbitrary",)),
    )(xf, gamma, beta, Wg)


def _ffn_body(x_ref, w1_ref, b1_ref, w2_ref, b2_ref, y_ref):
    f = pl.program_id(1)
    x = x_ref[0]
    h = jnp.dot(x, w1_ref[0], preferred_element_type=jnp.float32) + b1_ref[0]
    h = 0.5 * h * (1.0 + lax.erf(h * 0.7071067811865476))
    contrib = jnp.dot(h, w2_ref[0], preferred_element_type=jnp.float32)

    @pl.when(f == 0)
    def _():
        y_ref[0] = contrib

    @pl.when(f > 0)
    def _():
        y_ref[0] = y_ref[0] + contrib

    @pl.when(f == NF - 1)
    def _():
        y_ref[0] = y_ref[0] + b2_ref[0]


def _ffn_call(expert_in, W1, b1, W2, b2):
    return pl.pallas_call(
        _ffn_body,
        grid=(E, NF),
        in_specs=[
            pl.BlockSpec((1, C, H), lambda e, f: (e, 0, 0)),
            pl.BlockSpec((1, H, FB), lambda e, f: (e, 0, f)),
            pl.BlockSpec((1, 1, FB), lambda e, f: (e * NF + f, 0, 0)),
            pl.BlockSpec((1, FB, H), lambda e, f: (e, f, 0)),
            pl.BlockSpec((1, 1, H), lambda e, f: (e, 0, 0)),
        ],
        out_specs=pl.BlockSpec((1, C, H), lambda e, f: (e, 0, 0)),
        out_shape=jax.ShapeDtypeStruct((E, C, H), jnp.float32),
        compiler_params=pltpu.CompilerParams(
            dimension_semantics=("parallel", "arbitrary")),
    )(expert_in, W1, b1.reshape(E * NF, 1, FB), W2, b2.reshape(E, 1, H))


NC, NS, L = 2, 16, 16
NW = NC * NS
TPW = T // NW
SPW = (E * C) // NW
_MESH = plsc.VectorSubcoreMesh(core_axis_name="c", subcore_axis_name="s",
                               num_cores=NC, num_subcores=NS)


GCH = 32


def _dispatch_sc(routing, cnt0, dmy, normed):
    @functools.partial(
        pl.kernel,
        out_type=[
            jax.ShapeDtypeStruct((E * C, H), jnp.float32),
            jax.ShapeDtypeStruct((T,), jnp.int32),
            jax.ShapeDtypeStruct((T,), jnp.int32),
            jax.ShapeDtypeStruct((T,), jnp.float32),
            jax.ShapeDtypeStruct((T,), jnp.float32),
        ],
        mesh=_MESH,
        scratch_types=[
            pltpu.VMEM((E, T), jnp.float32),
            pltpu.VMEM((16,), jnp.int32),
            pltpu.VMEM((16,), jnp.int32),
            pltpu.VMEM((E * C,), jnp.int32),
            pltpu.VMEM((T,), jnp.int32),
            pltpu.VMEM((T,), jnp.float32),
            pltpu.VMEM_SHARED((E * C,), jnp.int32),
            pltpu.VMEM_SHARED((E * C,), jnp.int32),
            pltpu.VMEM((SPW,), jnp.int32),
            pltpu.VMEM((SPW,), jnp.int32),
            pltpu.VMEM((SPW,), jnp.int32),
            pltpu.VMEM((GCH, H), jnp.float32),
            pltpu.VMEM((GCH, H), jnp.float32),
            pltpu.SemaphoreType.DMA,
            pltpu.SemaphoreType.DMA,
            pltpu.SemaphoreType.DMA,
            pltpu.SemaphoreType.DMA,
        ],
        compiler_params=pltpu.CompilerParams(needs_layout_passes=False),
    )
    def body(rt_h, c0_h, dm_h, nm_h,
             ei_h, d0_h, d1_h, g0_h, g1_h,
             routv, cntv, dmyv, stv, dxv, gmv, shst0, shst1,
             st0v, st1v, idxv, rowsA, rowsB, semA, semB, semoA, semoB):
        cid = lax.axis_index("c")
        sid = lax.axis_index("s")
        wid = sid * NC + cid

        def phase1(krow, with_cnt, sh_ref, dx_hbm, gk_hbm):
            pltpu.sync_copy(rt_h, routv)
            pltpu.sync_copy(dm_h.at[0], dmyv)
            if with_cnt:
                pltpu.sync_copy(c0_h.at[0], cntv)

            def zinit(j, _):
                stv[pl.ds(j * L, L)] = jnp.zeros((L,), jnp.int32)
                return 0
            lax.fori_loop(0, (E * C) // L, zinit, 0)

            dmy16 = dmyv[...]

            def route(g, _):
                base = g * L
                tvec = lax.iota(jnp.int32, L) + base
                eg = routv[krow, pl.ds(base, L)].astype(jnp.int32)
                lg = routv[krow + 2, pl.ds(base, L)].astype(jnp.int32)
                gk = routv[krow + 4, pl.ds(base, L)]
                if with_cnt:
                    lg = lg + plsc.load_gather(cntv, [eg])
                d = eg * C + lg
                m = lg < C
                plsc.store_scatter(stv, [d], tvec, mask=m)
                dxv[pl.ds(base, L)] = jnp.where(m, d, dmy16)
                gmv[pl.ds(base, L)] = jnp.where(m, gk, 0.0)
                return 0
            lax.fori_loop(0, T // L, route, 0)

            pltpu.sync_copy(stv, sh_ref)

            @pl.when(cid == 0)
            def _():
                pltpu.sync_copy(dxv, dx_hbm)
                pltpu.sync_copy(gmv, gk_hbm)

        @pl.when(sid == 0)
        def _():
            phase1(0, False, shst0, d0_h, g0_h)

        @pl.when(sid == 1)
        def _():
            phase1(1, True, shst1, d1_h, g1_h)

        plsc.subcore_barrier()

        base = wid * SPW
        pltpu.sync_copy(shst0.at[pl.ds(base, SPW)], st0v)
        pltpu.sync_copy(shst1.at[pl.ds(base, SPW)], st1v)

        def merge(j, _):
            idxv[pl.ds(j * L, L)] = (st0v[pl.ds(j * L, L)]
                                     + st1v[pl.ds(j * L, L)])
            return 0
        lax.fori_loop(0, SPW // L, merge, 0)

        nck = SPW // GCH
        sets = [(rowsA, semA, semoA), (rowsB, semB, semoB)]
        incps, outcps = {}, {}

        def fire_in(ck):
            rows, sem, _ = sets[ck % 2]
            return pltpu.async_copy(nm_h.at[idxv.at[pl.ds(ck * GCH, GCH)]],
                                    rows, sem)

        incps[0] = fire_in(0)
        for ck in range(nck):
            rows, sem, semo = sets[ck % 2]
            incps[ck].wait()
            if ck + 1 < nck:
                if ck - 1 >= 0:
                    outcps[ck - 1].wait()
                incps[ck + 1] = fire_in(ck + 1)
            outcps[ck] = pltpu.async_copy(
                rows, ei_h.at[pl.ds(base + ck * GCH, GCH)], semo)
        for ck in range(max(0, nck - 2), nck):
            outcps[ck].wait()

    return body(routing, cnt0, dmy, normed)


def _combine_sc(xf, ys, d0m, d1m, g0m, g1m):
    CH = 8

    @functools.partial(
        pl.kernel,
        out_type=jax.ShapeDtypeStruct((T, H), jnp.float32),
        mesh=_MESH,
        scratch_types=(
            [pltpu.VMEM((TPW,), jnp.int32)] * 2
            + [pltpu.VMEM((16 + TPW,), jnp.float32)] * 2
            + [pltpu.VMEM((CH, H), jnp.float32)] * 9
            + [pltpu.SemaphoreType.DMA] * 6
        ),
        compiler_params=pltpu.CompilerParams(needs_layout_passes=False),
    )
    def body(x_h, ys_h, d0_h, d1_h, g0_h, g1_h, o_h, i0v, i1v, w0v, w1v,
             xv0, r00, r10, xv1, r01, r11, xv2, r02, r12,
             sin0, sin1, sin2, sout0, sout1, sout2):
        cid = lax.axis_index("c")
        sid = lax.axis_index("s")
        wid = sid * NC + cid
        base = wid * TPW
        pltpu.sync_copy(d0_h.at[pl.ds(base, TPW)], i0v)
        pltpu.sync_copy(d1_h.at[pl.ds(base, TPW)], i1v)
        pltpu.sync_copy(g0_h.at[pl.ds(base, TPW)], w0v.at[pl.ds(16, TPW)])
        pltpu.sync_copy(g1_h.at[pl.ds(base, TPW)], w1v.at[pl.ds(16, TPW)])

        sets = [(xv0, r00, r10, sin0, sout0),
                (xv1, r01, r11, sin1, sout1),
                (xv2, r02, r12, sin2, sout2)]
        nck = TPW // CH
        incps, outcps = {}, {}

        def fire_in(ck):
            xv, r0, r1, sin, _ = sets[ck % 3]
            tb = base + ck * CH
            return [
                pltpu.async_copy(x_h.at[pl.ds(tb, CH)], xv, sin),
                pltpu.async_copy(ys_h.at[i0v.at[pl.ds(ck * CH, CH)]], r0,
                                 sin),
                pltpu.async_copy(ys_h.at[i1v.at[pl.ds(ck * CH, CH)]], r1,
                                 sin),
            ]

        incps[0] = fire_in(0)
        incps[1] = fire_in(1)
        for ck in range(nck):
            xv, r0, r1, sin, sout = sets[ck % 3]
            for cp in incps[ck]:
                cp.wait()

            ws = [(plsc.load_gather(w0v, [jnp.full((L,), 16 + ck * CH + r,
                                                   jnp.int32)]),
                   plsc.load_gather(w1v, [jnp.full((L,), 16 + ck * CH + r,
                                                   jnp.int32)]))
                  for r in range(CH)]

            def col(j, _):
                for r in range(CH):
                    w0, w1 = ws[r]
                    xv[r, pl.ds(j * L, L)] = (xv[r, pl.ds(j * L, L)]
                                              + w0 * r0[r, pl.ds(j * L, L)]
                                              + w1 * r1[r, pl.ds(j * L, L)])
                return 0
            lax.fori_loop(0, H // L, col, 0)
            outcps[ck] = pltpu.async_copy(
                xv, o_h.at[pl.ds(base + ck * CH, CH)], sout)
            if ck + 2 < nck:
                if ck - 1 >= 0:
                    outcps[ck - 1].wait()
                incps[ck + 2] = fire_in(ck + 2)
        for ck in range(max(0, nck - 3), nck):
            outcps[ck].wait()

    return body(xf, ys, d0m, d1m, g0m, g1m)


def kernel(x, gamma, beta, Wg, W1, b1, W2, b2):
    xf = x.reshape(T, H)
    normed, routing, cnt0, dmy, counts2d, laux2d = _gate_call(
        xf, gamma.reshape(1, H), beta.reshape(1, H), Wg)
    counts = counts2d.reshape(E)
    l_aux = laux2d.reshape(())

    expert_in, d0m, d1m, g0m, g1m = _dispatch_sc(routing, cnt0, dmy, normed)
    ys = _ffn_call(expert_in.reshape(E, C, H), W1, b1, W2, b2
                   ).reshape(E * C, H)
    out_flat = _combine_sc(xf, ys, d0m, d1m, g0m, g1m)
    return out_flat.reshape(B, S, H), l_aux, counts
